# baseline (speedup 1.0000x reference)
import functools

import jax
import jax.numpy as jnp
from jax import lax
from jax.experimental import pallas as pl
from jax.experimental.pallas import tpu as pltpu
from jax.experimental.pallas import tpu_sc as plsc

N = 10000
C = 256
K = 27
NC = 512
DEPTH = 4

TN = 1000
NT = N // TN

_NCORE = 2
_NSUB = 16
_NW = _NCORE * _NSUB

_F32 = jnp.float32


def _sc_gather(table, idx, rows_per_iter):
    M = idx.shape[0]
    D = table.shape[1]
    b_per_w = M // _NW
    n_it = b_per_w // rows_per_iter
    assert M % _NW == 0 and b_per_w % rows_per_iter == 0
    assert rows_per_iter % 8 == 0

    mesh = plsc.VectorSubcoreMesh(core_axis_name="c", subcore_axis_name="s")
    R = rows_per_iter

    @functools.partial(
        pl.kernel,
        mesh=mesh,
        out_type=jax.ShapeDtypeStruct((M, D), _F32),
        scratch_types=[
            pltpu.VMEM((b_per_w,), jnp.int32),
            pltpu.VMEM((R, D), _F32),
            pltpu.VMEM((R, D), _F32),
            pltpu.SemaphoreType.DMA,
            pltpu.SemaphoreType.DMA,
            pltpu.SemaphoreType.DMA,
            pltpu.SemaphoreType.DMA,
        ],
    )
    def gather_kernel(table_hbm, idx_hbm, out_hbm, idx_v, buf0, buf1,
                      gsem0, gsem1, ssem0, ssem1):
        wid = lax.axis_index("s") * _NCORE + lax.axis_index("c")
        base = wid * b_per_w
        pltpu.sync_copy(idx_hbm.at[pl.ds(base, b_per_w)], idx_v)
        bufs = (buf0, buf1)
        gsems = (gsem0, gsem1)
        ssems = (ssem0, ssem1)

        def start_gather(i):
            return pltpu.async_copy(
                table_hbm.at[idx_v.at[pl.ds(i * R, R)]], bufs[i % 2],
                gsems[i % 2])

        def start_store(i):
            return pltpu.async_copy(
                bufs[i % 2], out_hbm.at[pl.ds(base + i * R, R)], ssems[i % 2])

        gathers = [None] * n_it
        stores = [None] * n_it
        gathers[0] = start_gather(0)
        for i in range(n_it):
            gathers[i].wait()
            stores[i] = start_store(i)
            if i + 1 < n_it:
                if i >= 1:
                    stores[i - 1].wait()
                gathers[i + 1] = start_gather(i + 1)
        stores[n_it - 1].wait()
        if n_it >= 2:
            stores[n_it - 2].wait()

    return gather_kernel(table, idx)


def _leaky(x):
    return jnp.where(x >= 0, x, 0.01 * x)


def _onehot(cl, nc):
    io = lax.broadcasted_iota(jnp.int32, (cl.shape[0], nc), 1)
    return (cl[:, None] == io).astype(_F32)


def _dot(a, b):
    return jnp.dot(a, b, preferred_element_type=_F32)


def _dotT(a, b):
    return lax.dot_general(a, b, (((0,), (0,)), ((), ())),
                           preferred_element_type=_F32)


def _bn_from_stats(t, stats, scale, bias, n):
    mu = stats[0:1] / n
    var = stats[1:2] / n - mu * mu
    return (t - mu) / jnp.sqrt(var + 1e-5) * scale + bias


def _mm_stats(x, w):
    B, _, D = w.shape

    def body(x_ref, w_ref, raw_ref, st_ref):
        j = pl.program_id(1)
        t = _dot(x_ref[...], w_ref[0])
        raw_ref[0] = t
        s = jnp.sum(t, axis=0, keepdims=True)
        q = jnp.sum(t * t, axis=0, keepdims=True)
        st = jnp.concatenate([s, q], axis=0)

        @pl.when(j == 0)
        def _():
            st_ref[0] = st

        @pl.when(j > 0)
        def _():
            st_ref[0] += st

    return pl.pallas_call(
        body,
        grid=(B, NT),
        in_specs=[
            pl.BlockSpec((TN, C), lambda b, j: (j, 0)),
            pl.BlockSpec((1, C, D), lambda b, j: (b, 0, 0)),
        ],
        out_specs=[
            pl.BlockSpec((1, TN, D), lambda b, j: (b, j, 0)),
            pl.BlockSpec((1, 2, D), lambda b, j: (b, 0, 0)),
        ],
        out_shape=[
            jax.ShapeDtypeStruct((B, N, D), _F32),
            jax.ShapeDtypeStruct((B, 2, D), _F32),
        ],
    )(x, w)


def _apply_seg(raw, stats, scale, bias, cls):
    def body(raw_ref, st_ref, sc_ref, bi_ref, cl_ref, a_ref, seg_ref, cnt_ref):
        j = pl.program_id(1)
        a = _leaky(_bn_from_stats(raw_ref[0], st_ref[0], sc_ref[0], bi_ref[0], N))
        a_ref[0] = a
        oh = _onehot(cl_ref[0, 0, 0], NC)
        seg = _dotT(oh, a)
        cnt = jnp.sum(oh, axis=0, keepdims=True)

        @pl.when(j == 0)
        def _():
            seg_ref[0] = seg
            cnt_ref[0] = cnt

        @pl.when(j > 0)
        def _():
            seg_ref[0] += seg
            cnt_ref[0] += cnt

    return pl.pallas_call(
        body,
        grid=(3, NT),
        in_specs=[
            pl.BlockSpec((1, TN, C), lambda b, j: (b, j, 0)),
            pl.BlockSpec((1, 2, C), lambda b, j: (b, 0, 0)),
            pl.BlockSpec((1, 1, C), lambda b, j: (b, 0, 0)),
            pl.BlockSpec((1, 1, C), lambda b, j: (b, 0, 0)),
            pl.BlockSpec((1, 1, 1, TN), lambda b, j: (b, j, 0, 0)),
        ],
        out_specs=[
            pl.BlockSpec((1, TN, C), lambda b, j: (b, j, 0)),
            pl.BlockSpec((1, NC, C), lambda b, j: (b, 0, 0)),
            pl.BlockSpec((1, 1, NC), lambda b, j: (b, 0, 0)),
        ],
        out_shape=[
            jax.ShapeDtypeStruct((3, N, C), _F32),
            jax.ShapeDtypeStruct((3, NC, C), _F32),
            jax.ShapeDtypeStruct((3, 1, NC), _F32),
        ],
    )(raw, stats, scale, bias, cls)


def _center_mm_max(a, segA, cnt, cls, w):
    def body(a_ref, seg_ref, cnt_ref, cl_ref, w_ref, b_ref, mx_ref):
        j = pl.program_id(1)
        oh = _onehot(cl_ref[0, 0, 0], NC)
        rc = 1.0 / jnp.maximum(cnt_ref[0], 1.0)
        m = _dot(oh * rc, seg_ref[0])
        bt = _dot(a_ref[0] - m, w_ref[0])
        b_ref[0] = bt
        tm = jnp.max(bt)

        @pl.when(j == 0)
        def _():
            mx_ref[...] = jnp.full((1, 1, NC), tm, _F32)

        @pl.when(j > 0)
        def _():
            mx_ref[...] = jnp.maximum(mx_ref[...], tm)

    return pl.pallas_call(
        body,
        grid=(3, NT),
        in_specs=[
            pl.BlockSpec((1, TN, C), lambda b, j: (b, j, 0)),
            pl.BlockSpec((1, NC, C), lambda b, j: (b, 0, 0)),
            pl.BlockSpec((1, 1, NC), lambda b, j: (b, 0, 0)),
            pl.BlockSpec((1, 1, 1, TN), lambda b, j: (b, j, 0, 0)),
            pl.BlockSpec((1, C, C), lambda b, j: (b, 0, 0)),
        ],
        out_specs=[
            pl.BlockSpec((1, TN, C), lambda b, j: (b, j, 0)),
            pl.BlockSpec((1, 1, NC), lambda b, j: (b, 0, 0)),
        ],
        out_shape=[
            jax.ShapeDtypeStruct((3, N, C), _F32),
            jax.ShapeDtypeStruct((3, 1, NC), _F32),
        ],
    )(a, segA, cnt, cls, w)


def _exp_seg(bmm, bmax, cls):
    def body(b_ref, mx_ref, cl_ref, e_ref, seg_ref):
        j = pl.program_id(1)
        e = jnp.exp(b_ref[0] - mx_ref[0, 0, 0])
        e_ref[0] = e
        oh = _onehot(cl_ref[0, 0, 0], NC)
        seg = _dotT(oh, e)

        @pl.when(j == 0)
        def _():
            seg_ref[0] = seg

        @pl.when(j > 0)
        def _():
            seg_ref[0] += seg

    return pl.pallas_call(
        body,
        grid=(3, NT),
        in_specs=[
            pl.BlockSpec((1, TN, C), lambda b, j: (b, j, 0)),
            pl.BlockSpec((1, 1, NC), lambda b, j: (b, 0, 0)),
            pl.BlockSpec((1, 1, 1, TN), lambda b, j: (b, j, 0, 0)),
        ],
        out_specs=[
            pl.BlockSpec((1, TN, C), lambda b, j: (b, j, 0)),
            pl.BlockSpec((1, NC, C), lambda b, j: (b, 0, 0)),
        ],
        out_shape=[
            jax.ShapeDtypeStruct((3, N, C), _F32),
            jax.ShapeDtypeStruct((3, NC, C), _F32),
        ],
    )(bmm, bmax, cls)


def _pw_seg(praw, pstats, pscale, pbias, e, segE, cls):
    def body(p_ref, st_ref, sc_ref, bi_ref, e_ref, se_ref, cl_ref, seg_ref):
        j = pl.program_id(1)
        p = _leaky(_bn_from_stats(p_ref[0], st_ref[0], sc_ref[0], bi_ref[0], N))
        oh = _onehot(cl_ref[0, 0, 0], NC)
        den = _dot(oh, se_ref[0]) + 1e-6
        pp = p * (e_ref[0] / den)
        seg = _dotT(oh, pp)

        @pl.when(j == 0)
        def _():
            seg_ref[0] = seg

        @pl.when(j > 0)
        def _():
            seg_ref[0] += seg

    return pl.pallas_call(
        body,
        grid=(3, NT),
        in_specs=[
            pl.BlockSpec((1, TN, C), lambda b, j: (b, j, 0)),
            pl.BlockSpec((1, 2, C), lambda b, j: (b, 0, 0)),
            pl.BlockSpec((1, 1, C), lambda b, j: (b, 0, 0)),
            pl.BlockSpec((1, 1, C), lambda b, j: (b, 0, 0)),
            pl.BlockSpec((1, TN, C), lambda b, j: (b, j, 0)),
            pl.BlockSpec((1, NC, C), lambda b, j: (b, 0, 0)),
            pl.BlockSpec((1, 1, 1, TN), lambda b, j: (b, j, 0, 0)),
        ],
        out_specs=[
            pl.BlockSpec((1, NC, C), lambda b, j: (b, 0, 0)),
        ],
        out_shape=[
            jax.ShapeDtypeStruct((3, NC, C), _F32),
        ],
    )(praw, pstats, pscale, pbias, e, segE, cls)[0]


def _fuse(feat, awp, msk, praw3, pstats3, pscale3, pbias3, segP, cls3, fw1, fw2):
    def body(x_ref, aw_ref, mk_ref, p_ref, st_ref, sc_ref, bi_ref, sp_ref,
             cl_ref, f1_ref, f2_ref, fr_ref, fst_ref):
        j = pl.program_id(0)
        logits = _dot(x_ref[...], aw_ref[0]) + mk_ref[0]
        z = logits - jnp.max(logits, axis=1, keepdims=True)
        ez = jnp.exp(z)
        adp = ez / jnp.sum(ez, axis=1, keepdims=True)
        fsum = jnp.zeros((TN, C), _F32)
        for b in range(3):
            oh = _onehot(cl_ref[b, 0, 0], NC)
            pf = _dot(oh, sp_ref[b])
            fsum = fsum + adp[:, b:b + 1] * pf
        f4 = _leaky(_bn_from_stats(p_ref[0], st_ref[0], sc_ref[0], bi_ref[0], N))
        fr = _dot(f4, f1_ref[...]) + _dot(fsum, f2_ref[...])
        fr_ref[...] = fr
        s = jnp.sum(fr, axis=0, keepdims=True)
        q = jnp.sum(fr * fr, axis=0, keepdims=True)
        st = jnp.concatenate([s, q], axis=0)

        @pl.when(j == 0)
        def _():
            fst_ref[0] = st

        @pl.when(j > 0)
        def _():
            fst_ref[0] += st

    return pl.pallas_call(
        body,
        grid=(NT,),
        in_specs=[
            pl.BlockSpec((TN, C), lambda j: (j, 0)),
            pl.BlockSpec((1, C, 128), lambda j: (0, 0, 0)),
            pl.BlockSpec((1, 128), lambda j: (0, 0)),
            pl.BlockSpec((1, TN, C), lambda j: (3, j, 0)),
            pl.BlockSpec((1, 2, C), lambda j: (3, 0, 0)),
            pl.BlockSpec((1, 1, C), lambda j: (0, 0, 0)),
            pl.BlockSpec((1, 1, C), lambda j: (0, 0, 0)),
            pl.BlockSpec((3, NC, C), lambda j: (0, 0, 0)),
            pl.BlockSpec((3, 1, 1, TN), lambda j: (0, j, 0, 0)),
            pl.BlockSpec((C, C), lambda j: (0, 0)),
            pl.BlockSpec((C, C), lambda j: (0, 0)),
        ],
        out_specs=[
            pl.BlockSpec((TN, C), lambda j: (j, 0)),
            pl.BlockSpec((1, 2, C), lambda j: (0, 0, 0)),
        ],
        out_shape=[
            jax.ShapeDtypeStruct((N, C), _F32),
            jax.ShapeDtypeStruct((1, 2, C), _F32),
        ],
    )(feat, awp, msk, praw3, pstats3, pscale3, pbias3, segP, cls3, fw1, fw2)


def _bn_res(raw, stats, scale, bias, res, leaky_first):
    def body(r_ref, st_ref, sc_ref, bi_ref, rs_ref, o_ref):
        t = _bn_from_stats(r_ref[...], st_ref[0], sc_ref[0], bi_ref[0], N)
        if leaky_first:
            o_ref[...] = _leaky(t) + rs_ref[...]
        else:
            o_ref[...] = _leaky(t + rs_ref[...])

    return pl.pallas_call(
        body,
        grid=(NT,),
        in_specs=[
            pl.BlockSpec((TN, C), lambda j: (j, 0)),
            pl.BlockSpec((1, 2, C), lambda j: (0, 0, 0)),
            pl.BlockSpec((1, 1, C), lambda j: (0, 0, 0)),
            pl.BlockSpec((1, 1, C), lambda j: (0, 0, 0)),
            pl.BlockSpec((TN, C), lambda j: (j, 0)),
        ],
        out_specs=[pl.BlockSpec((TN, C), lambda j: (j, 0))],
        out_shape=[jax.ShapeDtypeStruct((N, C), _F32)],
    )(raw, stats, scale, bias, res)[0]


def _bn_leaky(raw, stats, scale, bias):
    def body(r_ref, st_ref, sc_ref, bi_ref, o_ref):
        t = _bn_from_stats(r_ref[...], st_ref[0], sc_ref[0], bi_ref[0], N)
        o_ref[...] = _leaky(t)

    return pl.pallas_call(
        body,
        grid=(NT,),
        in_specs=[
            pl.BlockSpec((TN, C), lambda j: (j, 0)),
            pl.BlockSpec((1, 2, C), lambda j: (0, 0, 0)),
            pl.BlockSpec((1, 1, C), lambda j: (0, 0, 0)),
            pl.BlockSpec((1, 1, C), lambda j: (0, 0, 0)),
        ],
        out_specs=[pl.BlockSpec((TN, C), lambda j: (j, 0))],
        out_shape=[jax.ShapeDtypeStruct((N, C), _F32)],
    )(raw, stats, scale, bias)[0]


def _conv(g, w):
    def body(g_ref, w_ref, y_ref, st_ref):
        j = pl.program_id(0)
        k = pl.program_id(1)
        t = _dot(g_ref[...], w_ref[0])

        @pl.when(k == 0)
        def _():
            y_ref[...] = t

        @pl.when(k > 0)
        def _():
            y_ref[...] += t

        @pl.when(k == K - 1)
        def _():
            y = y_ref[...]
            s = jnp.sum(y, axis=0, keepdims=True)
            q = jnp.sum(y * y, axis=0, keepdims=True)
            st = jnp.concatenate([s, q], axis=0)

            @pl.when(j == 0)
            def _():
                st_ref[0] = st

            @pl.when(j > 0)
            def _():
                st_ref[0] += st

    return pl.pallas_call(
        body,
        grid=(NT, K),
        in_specs=[
            pl.BlockSpec((TN, C), lambda j, k: (k * NT + j, 0)),
            pl.BlockSpec((1, C, C), lambda j, k: (k, 0, 0)),
        ],
        out_specs=[
            pl.BlockSpec((TN, C), lambda j, k: (j, 0)),
            pl.BlockSpec((1, 2, C), lambda j, k: (0, 0, 0)),
        ],
        out_shape=[
            jax.ShapeDtypeStruct((N, C), _F32),
            jax.ShapeDtypeStruct((1, 2, C), _F32),
        ],
    )(g, w)


def kernel(feat, cluster0, cluster1, cluster2, neighbor_index, proj_w, proj_scale, proj_bias, lw_w, lw_scale, lw_bias, weight_w, adaptive_w, fuse_w, fuse_scale, fuse_bias, conv1_w, conv2_w, bn1_scale, bn1_bias, bn2_scale, bn2_bias):
    cls = jnp.stack([cluster0, cluster1, cluster2]).astype(jnp.int32)
    cls = cls.reshape(3, NT, 1, TN)

    a_raw, lw_st = _mm_stats(feat, lw_w)
    a, segA, cnt = _apply_seg(a_raw, lw_st,
                              lw_scale.reshape(3, 1, C), lw_bias.reshape(3, 1, C),
                              cls)
    bmm, bmax = _center_mm_max(a, segA, cnt, cls, weight_w)
    e, segE = _exp_seg(bmm, bmax, cls)

    p_raw, p_st = _mm_stats(feat, proj_w)
    segP = _pw_seg(p_raw[:3], p_st[:3],
                   proj_scale[:3].reshape(3, 1, C), proj_bias[:3].reshape(3, 1, C),
                   e, segE, cls)

    awp = jnp.pad(adaptive_w, ((0, 0), (0, 128 - (DEPTH - 1)))).reshape(1, C, 128)
    msk = jnp.where(jnp.arange(128) < DEPTH - 1, 0.0, -1e30)
    msk = msk.astype(_F32).reshape(1, 128)
    fuse_raw, fuse_st = _fuse(feat, awp, msk, p_raw, p_st,
                              proj_scale[3].reshape(1, 1, C),
                              proj_bias[3].reshape(1, 1, C),
                              segP, cls, fuse_w[:C], fuse_w[C:])
    f = _bn_res(fuse_raw, fuse_st,
                fuse_scale.reshape(1, 1, C), fuse_bias.reshape(1, 1, C),
                feat, leaky_first=True)

    MPAD = 288000
    nbrf = jnp.concatenate(
        [neighbor_index.reshape(-1).astype(jnp.int32),
         jnp.zeros((MPAD - K * N,), jnp.int32)]
    )

    g1 = _sc_gather(f, nbrf, 120)
    y_raw, y_st = _conv(g1, conv1_w)
    y = _bn_leaky(y_raw, y_st,
                  bn1_scale.reshape(1, 1, C), bn1_bias.reshape(1, 1, C))
    g2 = _sc_gather(y, nbrf, 120)
    y2_raw, y2_st = _conv(g2, conv2_w)
    out = _bn_res(y2_raw, y2_st,
                  bn2_scale.reshape(1, 1, C), bn2_bias.reshape(1, 1, C),
                  f, leaky_first=False)
    return out


# speedup vs baseline: 1.0295x; 1.0295x over previous
import functools

import jax
import jax.numpy as jnp
from jax import lax
from jax.experimental import pallas as pl
from jax.experimental.pallas import tpu as pltpu
from jax.experimental.pallas import tpu_sc as plsc

N = 10000
C = 256
K = 27
NC = 512
DEPTH = 4

TN = 1000
NT = N // TN

_NCORE = 2
_NSUB = 16
_NW = _NCORE * _NSUB

_F32 = jnp.float32


def _sc_gather(table, idx, rows_per_iter):
    M = idx.shape[0]
    D = table.shape[1]
    b_per_w = M // _NW
    n_it = b_per_w // rows_per_iter
    assert M % _NW == 0 and b_per_w % rows_per_iter == 0
    assert rows_per_iter % 8 == 0

    mesh = plsc.VectorSubcoreMesh(core_axis_name="c", subcore_axis_name="s")
    R = rows_per_iter
    NB = 4

    @functools.partial(
        pl.kernel,
        mesh=mesh,
        out_type=jax.ShapeDtypeStruct((M, D), _F32),
        scratch_types=[
            pltpu.VMEM((b_per_w,), jnp.int32),
            [pltpu.VMEM((R, D), _F32) for _ in range(NB)],
            [pltpu.SemaphoreType.DMA for _ in range(NB)],
            [pltpu.SemaphoreType.DMA for _ in range(NB)],
        ],
    )
    def gather_kernel(table_hbm, idx_hbm, out_hbm, idx_v, bufs, gsems, ssems):
        wid = lax.axis_index("s") * _NCORE + lax.axis_index("c")
        base = wid * b_per_w
        pltpu.sync_copy(idx_hbm.at[pl.ds(base, b_per_w)], idx_v)

        def start_gather(i):
            return pltpu.async_copy(
                table_hbm.at[idx_v.at[pl.ds(i * R, R)]], bufs[i % NB],
                gsems[i % NB])

        def start_store(i):
            return pltpu.async_copy(
                bufs[i % NB], out_hbm.at[pl.ds(base + i * R, R)], ssems[i % NB])

        gathers = [None] * n_it
        stores = [None] * n_it
        for i in range(min(NB - 1, n_it)):
            gathers[i] = start_gather(i)
        for i in range(n_it):
            nxt = i + NB - 1
            if nxt < n_it:
                if nxt - NB >= 0:
                    stores[nxt - NB].wait()
                gathers[nxt] = start_gather(nxt)
            gathers[i].wait()
            stores[i] = start_store(i)
        for i in range(max(0, n_it - NB), n_it):
            stores[i].wait()

    return gather_kernel(table, idx)


def _leaky(x):
    return jnp.where(x >= 0, x, 0.01 * x)


def _onehot(cl, nc):
    io = lax.broadcasted_iota(jnp.int32, (cl.shape[0], nc), 1)
    return (cl[:, None] == io).astype(_F32)


def _dot(a, b):
    return jnp.dot(a, b, preferred_element_type=_F32)


def _dotT(a, b):
    return lax.dot_general(a, b, (((0,), (0,)), ((), ())),
                           preferred_element_type=_F32)


def _bn_from_stats(t, stats, scale, bias, n):
    mu = stats[0:1] / n
    var = stats[1:2] / n - mu * mu
    return (t - mu) / jnp.sqrt(var + 1e-5) * scale + bias


def _mm_stats(x, w):
    B, _, D = w.shape

    def body(x_ref, w_ref, raw_ref, st_ref):
        j = pl.program_id(1)
        t = _dot(x_ref[...], w_ref[0])
        raw_ref[0] = t
        s = jnp.sum(t, axis=0, keepdims=True)
        q = jnp.sum(t * t, axis=0, keepdims=True)
        st = jnp.concatenate([s, q], axis=0)

        @pl.when(j == 0)
        def _():
            st_ref[0] = st

        @pl.when(j > 0)
        def _():
            st_ref[0] += st

    return pl.pallas_call(
        body,
        grid=(B, NT),
        in_specs=[
            pl.BlockSpec((TN, C), lambda b, j: (j, 0)),
            pl.BlockSpec((1, C, D), lambda b, j: (b, 0, 0)),
        ],
        out_specs=[
            pl.BlockSpec((1, TN, D), lambda b, j: (b, j, 0)),
            pl.BlockSpec((1, 2, D), lambda b, j: (b, 0, 0)),
        ],
        out_shape=[
            jax.ShapeDtypeStruct((B, N, D), _F32),
            jax.ShapeDtypeStruct((B, 2, D), _F32),
        ],
    )(x, w)


def _apply_seg(raw, stats, scale, bias, cls):
    def body(raw_ref, st_ref, sc_ref, bi_ref, cl_ref, a_ref, seg_ref, cnt_ref):
        j = pl.program_id(1)
        a = _leaky(_bn_from_stats(raw_ref[0], st_ref[0], sc_ref[0], bi_ref[0], N))
        a_ref[0] = a
        oh = _onehot(cl_ref[0, 0, 0], NC)
        seg = _dotT(oh, a)
        cnt = jnp.sum(oh, axis=0, keepdims=True)

        @pl.when(j == 0)
        def _():
            seg_ref[0] = seg
            cnt_ref[0] = cnt

        @pl.when(j > 0)
        def _():
            seg_ref[0] += seg
            cnt_ref[0] += cnt

    return pl.pallas_call(
        body,
        grid=(3, NT),
        in_specs=[
            pl.BlockSpec((1, TN, C), lambda b, j: (b, j, 0)),
            pl.BlockSpec((1, 2, C), lambda b, j: (b, 0, 0)),
            pl.BlockSpec((1, 1, C), lambda b, j: (b, 0, 0)),
            pl.BlockSpec((1, 1, C), lambda b, j: (b, 0, 0)),
            pl.BlockSpec((1, 1, 1, TN), lambda b, j: (b, j, 0, 0)),
        ],
        out_specs=[
            pl.BlockSpec((1, TN, C), lambda b, j: (b, j, 0)),
            pl.BlockSpec((1, NC, C), lambda b, j: (b, 0, 0)),
            pl.BlockSpec((1, 1, NC), lambda b, j: (b, 0, 0)),
        ],
        out_shape=[
            jax.ShapeDtypeStruct((3, N, C), _F32),
            jax.ShapeDtypeStruct((3, NC, C), _F32),
            jax.ShapeDtypeStruct((3, 1, NC), _F32),
        ],
    )(raw, stats, scale, bias, cls)


def _center_mm_max(a, segA, cnt, cls, w):
    def body(a_ref, seg_ref, cnt_ref, cl_ref, w_ref, b_ref, mx_ref):
        j = pl.program_id(1)
        oh = _onehot(cl_ref[0, 0, 0], NC)
        rc = 1.0 / jnp.maximum(cnt_ref[0], 1.0)
        m = _dot(oh * rc, seg_ref[0])
        bt = _dot(a_ref[0] - m, w_ref[0])
        b_ref[0] = bt
        tm = jnp.max(bt)

        @pl.when(j == 0)
        def _():
            mx_ref[...] = jnp.full((1, 1, NC), tm, _F32)

        @pl.when(j > 0)
        def _():
            mx_ref[...] = jnp.maximum(mx_ref[...], tm)

    return pl.pallas_call(
        body,
        grid=(3, NT),
        in_specs=[
            pl.BlockSpec((1, TN, C), lambda b, j: (b, j, 0)),
            pl.BlockSpec((1, NC, C), lambda b, j: (b, 0, 0)),
            pl.BlockSpec((1, 1, NC), lambda b, j: (b, 0, 0)),
            pl.BlockSpec((1, 1, 1, TN), lambda b, j: (b, j, 0, 0)),
            pl.BlockSpec((1, C, C), lambda b, j: (b, 0, 0)),
        ],
        out_specs=[
            pl.BlockSpec((1, TN, C), lambda b, j: (b, j, 0)),
            pl.BlockSpec((1, 1, NC), lambda b, j: (b, 0, 0)),
        ],
        out_shape=[
            jax.ShapeDtypeStruct((3, N, C), _F32),
            jax.ShapeDtypeStruct((3, 1, NC), _F32),
        ],
    )(a, segA, cnt, cls, w)


def _exp_seg(bmm, bmax, cls):
    def body(b_ref, mx_ref, cl_ref, e_ref, seg_ref):
        j = pl.program_id(1)
        e = jnp.exp(b_ref[0] - mx_ref[0, 0, 0])
        e_ref[0] = e
        oh = _onehot(cl_ref[0, 0, 0], NC)
        seg = _dotT(oh, e)

        @pl.when(j == 0)
        def _():
            seg_ref[0] = seg

        @pl.when(j > 0)
        def _():
            seg_ref[0] += seg

    return pl.pallas_call(
        body,
        grid=(3, NT),
        in_specs=[
            pl.BlockSpec((1, TN, C), lambda b, j: (b, j, 0)),
            pl.BlockSpec((1, 1, NC), lambda b, j: (b, 0, 0)),
            pl.BlockSpec((1, 1, 1, TN), lambda b, j: (b, j, 0, 0)),
        ],
        out_specs=[
            pl.BlockSpec((1, TN, C), lambda b, j: (b, j, 0)),
            pl.BlockSpec((1, NC, C), lambda b, j: (b, 0, 0)),
        ],
        out_shape=[
            jax.ShapeDtypeStruct((3, N, C), _F32),
            jax.ShapeDtypeStruct((3, NC, C), _F32),
        ],
    )(bmm, bmax, cls)


def _pw_seg(praw, pstats, pscale, pbias, e, segE, cls):
    def body(p_ref, st_ref, sc_ref, bi_ref, e_ref, se_ref, cl_ref, seg_ref):
        j = pl.program_id(1)
        p = _leaky(_bn_from_stats(p_ref[0], st_ref[0], sc_ref[0], bi_ref[0], N))
        oh = _onehot(cl_ref[0, 0, 0], NC)
        den = _dot(oh, se_ref[0]) + 1e-6
        pp = p * (e_ref[0] / den)
        seg = _dotT(oh, pp)

        @pl.when(j == 0)
        def _():
            seg_ref[0] = seg

        @pl.when(j > 0)
        def _():
            seg_ref[0] += seg

    return pl.pallas_call(
        body,
        grid=(3, NT),
        in_specs=[
            pl.BlockSpec((1, TN, C), lambda b, j: (b, j, 0)),
            pl.BlockSpec((1, 2, C), lambda b, j: (b, 0, 0)),
            pl.BlockSpec((1, 1, C), lambda b, j: (b, 0, 0)),
            pl.BlockSpec((1, 1, C), lambda b, j: (b, 0, 0)),
            pl.BlockSpec((1, TN, C), lambda b, j: (b, j, 0)),
            pl.BlockSpec((1, NC, C), lambda b, j: (b, 0, 0)),
            pl.BlockSpec((1, 1, 1, TN), lambda b, j: (b, j, 0, 0)),
        ],
        out_specs=[
            pl.BlockSpec((1, NC, C), lambda b, j: (b, 0, 0)),
        ],
        out_shape=[
            jax.ShapeDtypeStruct((3, NC, C), _F32),
        ],
    )(praw, pstats, pscale, pbias, e, segE, cls)[0]


def _fuse(feat, awp, msk, praw3, pstats3, pscale3, pbias3, segP, cls3, fw1, fw2):
    def body(x_ref, aw_ref, mk_ref, p_ref, st_ref, sc_ref, bi_ref, sp_ref,
             cl_ref, f1_ref, f2_ref, fr_ref, fst_ref):
        j = pl.program_id(0)
        logits = _dot(x_ref[...], aw_ref[0]) + mk_ref[0]
        z = logits - jnp.max(logits, axis=1, keepdims=True)
        ez = jnp.exp(z)
        adp = ez / jnp.sum(ez, axis=1, keepdims=True)
        fsum = jnp.zeros((TN, C), _F32)
        for b in range(3):
            oh = _onehot(cl_ref[b, 0, 0], NC)
            pf = _dot(oh, sp_ref[b])
            fsum = fsum + adp[:, b:b + 1] * pf
        f4 = _leaky(_bn_from_stats(p_ref[0], st_ref[0], sc_ref[0], bi_ref[0], N))
        fr = _dot(f4, f1_ref[...]) + _dot(fsum, f2_ref[...])
        fr_ref[...] = fr
        s = jnp.sum(fr, axis=0, keepdims=True)
        q = jnp.sum(fr * fr, axis=0, keepdims=True)
        st = jnp.concatenate([s, q], axis=0)

        @pl.when(j == 0)
        def _():
            fst_ref[0] = st

        @pl.when(j > 0)
        def _():
            fst_ref[0] += st

    return pl.pallas_call(
        body,
        grid=(NT,),
        in_specs=[
            pl.BlockSpec((TN, C), lambda j: (j, 0)),
            pl.BlockSpec((1, C, 128), lambda j: (0, 0, 0)),
            pl.BlockSpec((1, 128), lambda j: (0, 0)),
            pl.BlockSpec((1, TN, C), lambda j: (3, j, 0)),
            pl.BlockSpec((1, 2, C), lambda j: (3, 0, 0)),
            pl.BlockSpec((1, 1, C), lambda j: (0, 0, 0)),
            pl.BlockSpec((1, 1, C), lambda j: (0, 0, 0)),
            pl.BlockSpec((3, NC, C), lambda j: (0, 0, 0)),
            pl.BlockSpec((3, 1, 1, TN), lambda j: (0, j, 0, 0)),
            pl.BlockSpec((C, C), lambda j: (0, 0)),
            pl.BlockSpec((C, C), lambda j: (0, 0)),
        ],
        out_specs=[
            pl.BlockSpec((TN, C), lambda j: (j, 0)),
            pl.BlockSpec((1, 2, C), lambda j: (0, 0, 0)),
        ],
        out_shape=[
            jax.ShapeDtypeStruct((N, C), _F32),
            jax.ShapeDtypeStruct((1, 2, C), _F32),
        ],
    )(feat, awp, msk, praw3, pstats3, pscale3, pbias3, segP, cls3, fw1, fw2)


def _bn_res(raw, stats, scale, bias, res, leaky_first):
    def body(r_ref, st_ref, sc_ref, bi_ref, rs_ref, o_ref):
        t = _bn_from_stats(r_ref[...], st_ref[0], sc_ref[0], bi_ref[0], N)
        if leaky_first:
            o_ref[...] = _leaky(t) + rs_ref[...]
        else:
            o_ref[...] = _leaky(t + rs_ref[...])

    return pl.pallas_call(
        body,
        grid=(NT,),
        in_specs=[
            pl.BlockSpec((TN, C), lambda j: (j, 0)),
            pl.BlockSpec((1, 2, C), lambda j: (0, 0, 0)),
            pl.BlockSpec((1, 1, C), lambda j: (0, 0, 0)),
            pl.BlockSpec((1, 1, C), lambda j: (0, 0, 0)),
            pl.BlockSpec((TN, C), lambda j: (j, 0)),
        ],
        out_specs=[pl.BlockSpec((TN, C), lambda j: (j, 0))],
        out_shape=[jax.ShapeDtypeStruct((N, C), _F32)],
    )(raw, stats, scale, bias, res)[0]


def _bn_leaky(raw, stats, scale, bias):
    def body(r_ref, st_ref, sc_ref, bi_ref, o_ref):
        t = _bn_from_stats(r_ref[...], st_ref[0], sc_ref[0], bi_ref[0], N)
        o_ref[...] = _leaky(t)

    return pl.pallas_call(
        body,
        grid=(NT,),
        in_specs=[
            pl.BlockSpec((TN, C), lambda j: (j, 0)),
            pl.BlockSpec((1, 2, C), lambda j: (0, 0, 0)),
            pl.BlockSpec((1, 1, C), lambda j: (0, 0, 0)),
            pl.BlockSpec((1, 1, C), lambda j: (0, 0, 0)),
        ],
        out_specs=[pl.BlockSpec((TN, C), lambda j: (j, 0))],
        out_shape=[jax.ShapeDtypeStruct((N, C), _F32)],
    )(raw, stats, scale, bias)[0]


def _conv(g, w):
    def body(g_ref, w_ref, y_ref, st_ref):
        j = pl.program_id(0)
        k = pl.program_id(1)
        t = _dot(g_ref[...], w_ref[0])

        @pl.when(k == 0)
        def _():
            y_ref[...] = t

        @pl.when(k > 0)
        def _():
            y_ref[...] += t

        @pl.when(k == K - 1)
        def _():
            y = y_ref[...]
            s = jnp.sum(y, axis=0, keepdims=True)
            q = jnp.sum(y * y, axis=0, keepdims=True)
            st = jnp.concatenate([s, q], axis=0)

            @pl.when(j == 0)
            def _():
                st_ref[0] = st

            @pl.when(j > 0)
            def _():
                st_ref[0] += st

    return pl.pallas_call(
        body,
        grid=(NT, K),
        in_specs=[
            pl.BlockSpec((TN, C), lambda j, k: (k * NT + j, 0)),
            pl.BlockSpec((1, C, C), lambda j, k: (k, 0, 0)),
        ],
        out_specs=[
            pl.BlockSpec((TN, C), lambda j, k: (j, 0)),
            pl.BlockSpec((1, 2, C), lambda j, k: (0, 0, 0)),
        ],
        out_shape=[
            jax.ShapeDtypeStruct((N, C), _F32),
            jax.ShapeDtypeStruct((1, 2, C), _F32),
        ],
    )(g, w)


def kernel(feat, cluster0, cluster1, cluster2, neighbor_index, proj_w, proj_scale, proj_bias, lw_w, lw_scale, lw_bias, weight_w, adaptive_w, fuse_w, fuse_scale, fuse_bias, conv1_w, conv2_w, bn1_scale, bn1_bias, bn2_scale, bn2_bias):
    cls = jnp.stack([cluster0, cluster1, cluster2]).astype(jnp.int32)
    cls = cls.reshape(3, NT, 1, TN)

    a_raw, lw_st = _mm_stats(feat, lw_w)
    a, segA, cnt = _apply_seg(a_raw, lw_st,
                              lw_scale.reshape(3, 1, C), lw_bias.reshape(3, 1, C),
                              cls)
    bmm, bmax = _center_mm_max(a, segA, cnt, cls, weight_w)
    e, segE = _exp_seg(bmm, bmax, cls)

    p_raw, p_st = _mm_stats(feat, proj_w)
    segP = _pw_seg(p_raw[:3], p_st[:3],
                   proj_scale[:3].reshape(3, 1, C), proj_bias[:3].reshape(3, 1, C),
                   e, segE, cls)

    awp = jnp.pad(adaptive_w, ((0, 0), (0, 128 - (DEPTH - 1)))).reshape(1, C, 128)
    msk = jnp.where(jnp.arange(128) < DEPTH - 1, 0.0, -1e30)
    msk = msk.astype(_F32).reshape(1, 128)
    fuse_raw, fuse_st = _fuse(feat, awp, msk, p_raw, p_st,
                              proj_scale[3].reshape(1, 1, C),
                              proj_bias[3].reshape(1, 1, C),
                              segP, cls, fuse_w[:C], fuse_w[C:])
    f = _bn_res(fuse_raw, fuse_st,
                fuse_scale.reshape(1, 1, C), fuse_bias.reshape(1, 1, C),
                feat, leaky_first=True)

    MPAD = 288000
    nbrf = jnp.concatenate(
        [neighbor_index.reshape(-1).astype(jnp.int32),
         jnp.zeros((MPAD - K * N,), jnp.int32)]
    )

    g1 = _sc_gather(f, nbrf, 72)
    y_raw, y_st = _conv(g1, conv1_w)
    y = _bn_leaky(y_raw, y_st,
                  bn1_scale.reshape(1, 1, C), bn1_bias.reshape(1, 1, C))
    g2 = _sc_gather(y, nbrf, 72)
    y2_raw, y2_st = _conv(g2, conv2_w)
    out = _bn_res(y2_raw, y2_st,
                  bn2_scale.reshape(1, 1, C), bn2_bias.reshape(1, 1, C),
                  f, leaky_first=False)
    return out


# speedup vs baseline: 1.1627x; 1.1294x over previous
import functools

import jax
import jax.numpy as jnp
from jax import lax
from jax.experimental import pallas as pl
from jax.experimental.pallas import tpu as pltpu
from jax.experimental.pallas import tpu_sc as plsc

N = 10000
C = 256
K = 27
NC = 512
DEPTH = 4

TN = 1000
NT = N // TN

_NCORE = 2
_NSUB = 16
_NW = _NCORE * _NSUB

_F32 = jnp.float32


def _sc_gather(table, idx, rows_per_iter):
    M = idx.shape[0]
    D = table.shape[1]
    b_per_w = M // _NW
    n_it = b_per_w // rows_per_iter
    assert M % _NW == 0 and b_per_w % rows_per_iter == 0
    assert rows_per_iter % 8 == 0

    mesh = plsc.VectorSubcoreMesh(core_axis_name="c", subcore_axis_name="s")
    R = rows_per_iter
    NB = 4

    @functools.partial(
        pl.kernel,
        mesh=mesh,
        out_type=jax.ShapeDtypeStruct((M, D), table.dtype),
        scratch_types=[
            pltpu.VMEM((b_per_w,), jnp.int32),
            [pltpu.VMEM((R, D), table.dtype) for _ in range(NB)],
            [pltpu.SemaphoreType.DMA for _ in range(NB)],
            [pltpu.SemaphoreType.DMA for _ in range(NB)],
        ],
    )
    def gather_kernel(table_hbm, idx_hbm, out_hbm, idx_v, bufs, gsems, ssems):
        wid = lax.axis_index("s") * _NCORE + lax.axis_index("c")
        base = wid * b_per_w
        pltpu.sync_copy(idx_hbm.at[pl.ds(base, b_per_w)], idx_v)

        def start_gather(i):
            return pltpu.async_copy(
                table_hbm.at[idx_v.at[pl.ds(i * R, R)]], bufs[i % NB],
                gsems[i % NB])

        def start_store(i):
            return pltpu.async_copy(
                bufs[i % NB], out_hbm.at[pl.ds(base + i * R, R)], ssems[i % NB])

        gathers = [None] * n_it
        stores = [None] * n_it
        for i in range(min(NB - 1, n_it)):
            gathers[i] = start_gather(i)
        for i in range(n_it):
            nxt = i + NB - 1
            if nxt < n_it:
                if nxt - NB >= 0:
                    stores[nxt - NB].wait()
                gathers[nxt] = start_gather(nxt)
            gathers[i].wait()
            stores[i] = start_store(i)
        for i in range(max(0, n_it - NB), n_it):
            stores[i].wait()

    return gather_kernel(table, idx)


def _leaky(x):
    return jnp.where(x >= 0, x, 0.01 * x)


def _onehot(cl, nc):
    io = lax.broadcasted_iota(jnp.int32, (cl.shape[0], nc), 1)
    return (cl[:, None] == io).astype(_F32)


def _dot(a, b):
    return jnp.dot(a, b, preferred_element_type=_F32)


def _dotT(a, b):
    return lax.dot_general(a, b, (((0,), (0,)), ((), ())),
                           preferred_element_type=_F32)


def _bn_from_stats(t, stats, scale, bias, n):
    mu = stats[0:1] / n
    var = stats[1:2] / n - mu * mu
    return (t - mu) / jnp.sqrt(var + 1e-5) * scale + bias


def _pack_bf16(t):
    tb = t.astype(jnp.bfloat16)
    lo = lax.bitcast_convert_type(tb[:, :128], jnp.uint16).astype(jnp.int32)
    hi = lax.bitcast_convert_type(tb[:, 128:], jnp.uint16).astype(jnp.int32)
    return lo | lax.shift_left(hi, 16)


def _unpack_bf16(p):
    lo = lax.bitcast_convert_type((p & 0xFFFF).astype(jnp.uint16), jnp.bfloat16)
    hi = lax.bitcast_convert_type(
        lax.shift_right_logical(p, 16).astype(jnp.uint16), jnp.bfloat16)
    return lo, hi


def _mm_stats(x, w):
    B, _, D = w.shape

    def body(x_ref, w_ref, raw_ref, st_ref):
        j = pl.program_id(1)
        t = _dot(x_ref[...], w_ref[0])
        raw_ref[0] = t
        s = jnp.sum(t, axis=0, keepdims=True)
        q = jnp.sum(t * t, axis=0, keepdims=True)
        st = jnp.concatenate([s, q], axis=0)

        @pl.when(j == 0)
        def _():
            st_ref[0] = st

        @pl.when(j > 0)
        def _():
            st_ref[0] += st

    return pl.pallas_call(
        body,
        grid=(B, NT),
        in_specs=[
            pl.BlockSpec((TN, C), lambda b, j: (j, 0)),
            pl.BlockSpec((1, C, D), lambda b, j: (b, 0, 0)),
        ],
        out_specs=[
            pl.BlockSpec((1, TN, D), lambda b, j: (b, j, 0)),
            pl.BlockSpec((1, 2, D), lambda b, j: (b, 0, 0)),
        ],
        out_shape=[
            jax.ShapeDtypeStruct((B, N, D), _F32),
            jax.ShapeDtypeStruct((B, 2, D), _F32),
        ],
    )(x, w)


def _apply_seg(raw, stats, scale, bias, cls):
    def body(raw_ref, st_ref, sc_ref, bi_ref, cl_ref, a_ref, seg_ref, cnt_ref):
        j = pl.program_id(1)
        a = _leaky(_bn_from_stats(raw_ref[0], st_ref[0], sc_ref[0], bi_ref[0], N))
        a_ref[0] = a
        oh = _onehot(cl_ref[0, 0, 0], NC)
        seg = _dotT(oh, a)
        cnt = jnp.sum(oh, axis=0, keepdims=True)

        @pl.when(j == 0)
        def _():
            seg_ref[0] = seg
            cnt_ref[0] = cnt

        @pl.when(j > 0)
        def _():
            seg_ref[0] += seg
            cnt_ref[0] += cnt

    return pl.pallas_call(
        body,
        grid=(3, NT),
        in_specs=[
            pl.BlockSpec((1, TN, C), lambda b, j: (b, j, 0)),
            pl.BlockSpec((1, 2, C), lambda b, j: (b, 0, 0)),
            pl.BlockSpec((1, 1, C), lambda b, j: (b, 0, 0)),
            pl.BlockSpec((1, 1, C), lambda b, j: (b, 0, 0)),
            pl.BlockSpec((1, 1, 1, TN), lambda b, j: (b, j, 0, 0)),
        ],
        out_specs=[
            pl.BlockSpec((1, TN, C), lambda b, j: (b, j, 0)),
            pl.BlockSpec((1, NC, C), lambda b, j: (b, 0, 0)),
            pl.BlockSpec((1, 1, NC), lambda b, j: (b, 0, 0)),
        ],
        out_shape=[
            jax.ShapeDtypeStruct((3, N, C), _F32),
            jax.ShapeDtypeStruct((3, NC, C), _F32),
            jax.ShapeDtypeStruct((3, 1, NC), _F32),
        ],
    )(raw, stats, scale, bias, cls)


def _center_mm_max(a, segA, cnt, cls, w):
    def body(a_ref, seg_ref, cnt_ref, cl_ref, w_ref, b_ref, mx_ref):
        j = pl.program_id(1)
        oh = _onehot(cl_ref[0, 0, 0], NC)
        rc = 1.0 / jnp.maximum(cnt_ref[0], 1.0)
        m = _dot(oh * rc, seg_ref[0])
        bt = _dot(a_ref[0] - m, w_ref[0])
        b_ref[0] = bt
        tm = jnp.max(bt)

        @pl.when(j == 0)
        def _():
            mx_ref[...] = jnp.full((1, 1, NC), tm, _F32)

        @pl.when(j > 0)
        def _():
            mx_ref[...] = jnp.maximum(mx_ref[...], tm)

    return pl.pallas_call(
        body,
        grid=(3, NT),
        in_specs=[
            pl.BlockSpec((1, TN, C), lambda b, j: (b, j, 0)),
            pl.BlockSpec((1, NC, C), lambda b, j: (b, 0, 0)),
            pl.BlockSpec((1, 1, NC), lambda b, j: (b, 0, 0)),
            pl.BlockSpec((1, 1, 1, TN), lambda b, j: (b, j, 0, 0)),
            pl.BlockSpec((1, C, C), lambda b, j: (b, 0, 0)),
        ],
        out_specs=[
            pl.BlockSpec((1, TN, C), lambda b, j: (b, j, 0)),
            pl.BlockSpec((1, 1, NC), lambda b, j: (b, 0, 0)),
        ],
        out_shape=[
            jax.ShapeDtypeStruct((3, N, C), _F32),
            jax.ShapeDtypeStruct((3, 1, NC), _F32),
        ],
    )(a, segA, cnt, cls, w)


def _exp_seg(bmm, bmax, cls):
    def body(b_ref, mx_ref, cl_ref, e_ref, seg_ref):
        j = pl.program_id(1)
        e = jnp.exp(b_ref[0] - mx_ref[0, 0, 0])
        e_ref[0] = e
        oh = _onehot(cl_ref[0, 0, 0], NC)
        seg = _dotT(oh, e)

        @pl.when(j == 0)
        def _():
            seg_ref[0] = seg

        @pl.when(j > 0)
        def _():
            seg_ref[0] += seg

    return pl.pallas_call(
        body,
        grid=(3, NT),
        in_specs=[
            pl.BlockSpec((1, TN, C), lambda b, j: (b, j, 0)),
            pl.BlockSpec((1, 1, NC), lambda b, j: (b, 0, 0)),
            pl.BlockSpec((1, 1, 1, TN), lambda b, j: (b, j, 0, 0)),
        ],
        out_specs=[
            pl.BlockSpec((1, TN, C), lambda b, j: (b, j, 0)),
            pl.BlockSpec((1, NC, C), lambda b, j: (b, 0, 0)),
        ],
        out_shape=[
            jax.ShapeDtypeStruct((3, N, C), _F32),
            jax.ShapeDtypeStruct((3, NC, C), _F32),
        ],
    )(bmm, bmax, cls)


def _pw_seg(praw, pstats, pscale, pbias, e, segE, cls):
    def body(p_ref, st_ref, sc_ref, bi_ref, e_ref, se_ref, cl_ref, seg_ref):
        j = pl.program_id(1)
        p = _leaky(_bn_from_stats(p_ref[0], st_ref[0], sc_ref[0], bi_ref[0], N))
        oh = _onehot(cl_ref[0, 0, 0], NC)
        den = _dot(oh, se_ref[0]) + 1e-6
        pp = p * (e_ref[0] / den)
        seg = _dotT(oh, pp)

        @pl.when(j == 0)
        def _():
            seg_ref[0] = seg

        @pl.when(j > 0)
        def _():
            seg_ref[0] += seg

    return pl.pallas_call(
        body,
        grid=(3, NT),
        in_specs=[
            pl.BlockSpec((1, TN, C), lambda b, j: (b, j, 0)),
            pl.BlockSpec((1, 2, C), lambda b, j: (b, 0, 0)),
            pl.BlockSpec((1, 1, C), lambda b, j: (b, 0, 0)),
            pl.BlockSpec((1, 1, C), lambda b, j: (b, 0, 0)),
            pl.BlockSpec((1, TN, C), lambda b, j: (b, j, 0)),
            pl.BlockSpec((1, NC, C), lambda b, j: (b, 0, 0)),
            pl.BlockSpec((1, 1, 1, TN), lambda b, j: (b, j, 0, 0)),
        ],
        out_specs=[
            pl.BlockSpec((1, NC, C), lambda b, j: (b, 0, 0)),
        ],
        out_shape=[
            jax.ShapeDtypeStruct((3, NC, C), _F32),
        ],
    )(praw, pstats, pscale, pbias, e, segE, cls)[0]


def _fuse(feat, awp, msk, praw3, pstats3, pscale3, pbias3, segP, cls3, fw1, fw2):
    def body(x_ref, aw_ref, mk_ref, p_ref, st_ref, sc_ref, bi_ref, sp_ref,
             cl_ref, f1_ref, f2_ref, fr_ref, fst_ref):
        j = pl.program_id(0)
        logits = _dot(x_ref[...], aw_ref[0]) + mk_ref[0]
        z = logits - jnp.max(logits, axis=1, keepdims=True)
        ez = jnp.exp(z)
        adp = ez / jnp.sum(ez, axis=1, keepdims=True)
        fsum = jnp.zeros((TN, C), _F32)
        for b in range(3):
            oh = _onehot(cl_ref[b, 0, 0], NC)
            pf = _dot(oh, sp_ref[b])
            fsum = fsum + adp[:, b:b + 1] * pf
        f4 = _leaky(_bn_from_stats(p_ref[0], st_ref[0], sc_ref[0], bi_ref[0], N))
        fr = _dot(f4, f1_ref[...]) + _dot(fsum, f2_ref[...])
        fr_ref[...] = fr
        s = jnp.sum(fr, axis=0, keepdims=True)
        q = jnp.sum(fr * fr, axis=0, keepdims=True)
        st = jnp.concatenate([s, q], axis=0)

        @pl.when(j == 0)
        def _():
            fst_ref[0] = st

        @pl.when(j > 0)
        def _():
            fst_ref[0] += st

    return pl.pallas_call(
        body,
        grid=(NT,),
        in_specs=[
            pl.BlockSpec((TN, C), lambda j: (j, 0)),
            pl.BlockSpec((1, C, 128), lambda j: (0, 0, 0)),
            pl.BlockSpec((1, 128), lambda j: (0, 0)),
            pl.BlockSpec((1, TN, C), lambda j: (3, j, 0)),
            pl.BlockSpec((1, 2, C), lambda j: (3, 0, 0)),
            pl.BlockSpec((1, 1, C), lambda j: (0, 0, 0)),
            pl.BlockSpec((1, 1, C), lambda j: (0, 0, 0)),
            pl.BlockSpec((3, NC, C), lambda j: (0, 0, 0)),
            pl.BlockSpec((3, 1, 1, TN), lambda j: (0, j, 0, 0)),
            pl.BlockSpec((C, C), lambda j: (0, 0)),
            pl.BlockSpec((C, C), lambda j: (0, 0)),
        ],
        out_specs=[
            pl.BlockSpec((TN, C), lambda j: (j, 0)),
            pl.BlockSpec((1, 2, C), lambda j: (0, 0, 0)),
        ],
        out_shape=[
            jax.ShapeDtypeStruct((N, C), _F32),
            jax.ShapeDtypeStruct((1, 2, C), _F32),
        ],
    )(feat, awp, msk, praw3, pstats3, pscale3, pbias3, segP, cls3, fw1, fw2)


def _bn_res(raw, stats, scale, bias, res, leaky_first, packed=False):
    def body(r_ref, st_ref, sc_ref, bi_ref, rs_ref, o_ref, *pk):
        t = _bn_from_stats(r_ref[...], st_ref[0], sc_ref[0], bi_ref[0], N)
        if leaky_first:
            t = _leaky(t) + rs_ref[...]
        else:
            t = _leaky(t + rs_ref[...])
        o_ref[...] = t
        if packed:
            pk[0][...] = _pack_bf16(t)

    out_specs = [pl.BlockSpec((TN, C), lambda j: (j, 0))]
    out_shape = [jax.ShapeDtypeStruct((N, C), _F32)]
    if packed:
        out_specs.append(pl.BlockSpec((TN, C // 2), lambda j: (j, 0)))
        out_shape.append(jax.ShapeDtypeStruct((N, C // 2), jnp.int32))
    r = pl.pallas_call(
        body,
        grid=(NT,),
        in_specs=[
            pl.BlockSpec((TN, C), lambda j: (j, 0)),
            pl.BlockSpec((1, 2, C), lambda j: (0, 0, 0)),
            pl.BlockSpec((1, 1, C), lambda j: (0, 0, 0)),
            pl.BlockSpec((1, 1, C), lambda j: (0, 0, 0)),
            pl.BlockSpec((TN, C), lambda j: (j, 0)),
        ],
        out_specs=out_specs,
        out_shape=out_shape,
    )(raw, stats, scale, bias, res)
    return r if packed else r[0]


def _bn_leaky_packed(raw, stats, scale, bias):
    def body(r_ref, st_ref, sc_ref, bi_ref, o_ref):
        t = _bn_from_stats(r_ref[...], st_ref[0], sc_ref[0], bi_ref[0], N)
        o_ref[...] = _pack_bf16(_leaky(t))

    return pl.pallas_call(
        body,
        grid=(NT,),
        in_specs=[
            pl.BlockSpec((TN, C), lambda j: (j, 0)),
            pl.BlockSpec((1, 2, C), lambda j: (0, 0, 0)),
            pl.BlockSpec((1, 1, C), lambda j: (0, 0, 0)),
            pl.BlockSpec((1, 1, C), lambda j: (0, 0, 0)),
        ],
        out_specs=[pl.BlockSpec((TN, C // 2), lambda j: (j, 0))],
        out_shape=[jax.ShapeDtypeStruct((N, C // 2), jnp.int32)],
    )(raw, stats, scale, bias)[0]


def _conv(g, w):
    def body(g_ref, w_ref, y_ref, st_ref):
        j = pl.program_id(0)
        k = pl.program_id(1)
        glo, ghi = _unpack_bf16(g_ref[...])
        t = (_dot(glo, w_ref[0, :C // 2]) + _dot(ghi, w_ref[0, C // 2:]))

        @pl.when(k == 0)
        def _():
            y_ref[...] = t

        @pl.when(k > 0)
        def _():
            y_ref[...] += t

        @pl.when(k == K - 1)
        def _():
            y = y_ref[...]
            s = jnp.sum(y, axis=0, keepdims=True)
            q = jnp.sum(y * y, axis=0, keepdims=True)
            st = jnp.concatenate([s, q], axis=0)

            @pl.when(j == 0)
            def _():
                st_ref[0] = st

            @pl.when(j > 0)
            def _():
                st_ref[0] += st

    return pl.pallas_call(
        body,
        grid=(NT, K),
        in_specs=[
            pl.BlockSpec((TN, C // 2), lambda j, k: (k * NT + j, 0)),
            pl.BlockSpec((1, C, C), lambda j, k: (k, 0, 0)),
        ],
        out_specs=[
            pl.BlockSpec((TN, C), lambda j, k: (j, 0)),
            pl.BlockSpec((1, 2, C), lambda j, k: (0, 0, 0)),
        ],
        out_shape=[
            jax.ShapeDtypeStruct((N, C), _F32),
            jax.ShapeDtypeStruct((1, 2, C), _F32),
        ],
    )(g, w)


def kernel(feat, cluster0, cluster1, cluster2, neighbor_index, proj_w, proj_scale, proj_bias, lw_w, lw_scale, lw_bias, weight_w, adaptive_w, fuse_w, fuse_scale, fuse_bias, conv1_w, conv2_w, bn1_scale, bn1_bias, bn2_scale, bn2_bias):
    cls = jnp.stack([cluster0, cluster1, cluster2]).astype(jnp.int32)
    cls = cls.reshape(3, NT, 1, TN)

    a_raw, lw_st = _mm_stats(feat, lw_w)
    a, segA, cnt = _apply_seg(a_raw, lw_st,
                              lw_scale.reshape(3, 1, C), lw_bias.reshape(3, 1, C),
                              cls)
    bmm, bmax = _center_mm_max(a, segA, cnt, cls, weight_w)
    e, segE = _exp_seg(bmm, bmax, cls)

    p_raw, p_st = _mm_stats(feat, proj_w)
    segP = _pw_seg(p_raw[:3], p_st[:3],
                   proj_scale[:3].reshape(3, 1, C), proj_bias[:3].reshape(3, 1, C),
                   e, segE, cls)

    awp = jnp.pad(adaptive_w, ((0, 0), (0, 128 - (DEPTH - 1)))).reshape(1, C, 128)
    msk = jnp.where(jnp.arange(128) < DEPTH - 1, 0.0, -1e30)
    msk = msk.astype(_F32).reshape(1, 128)
    fuse_raw, fuse_st = _fuse(feat, awp, msk, p_raw, p_st,
                              proj_scale[3].reshape(1, 1, C),
                              proj_bias[3].reshape(1, 1, C),
                              segP, cls, fuse_w[:C], fuse_w[C:])
    f, f_pk = _bn_res(fuse_raw, fuse_st,
                      fuse_scale.reshape(1, 1, C), fuse_bias.reshape(1, 1, C),
                      feat, leaky_first=True, packed=True)

    MPAD = 288000
    nbrf = jnp.concatenate(
        [neighbor_index.reshape(-1).astype(jnp.int32),
         jnp.zeros((MPAD - K * N,), jnp.int32)]
    )
    c1w = conv1_w.astype(jnp.bfloat16)
    c2w = conv2_w.astype(jnp.bfloat16)

    g1 = _sc_gather(f_pk, nbrf, 120)
    y_raw, y_st = _conv(g1, c1w)
    y_pk = _bn_leaky_packed(y_raw, y_st,
                            bn1_scale.reshape(1, 1, C), bn1_bias.reshape(1, 1, C))
    g2 = _sc_gather(y_pk, nbrf, 120)
    y2_raw, y2_st = _conv(g2, c2w)
    out = _bn_res(y2_raw, y2_st,
                  bn2_scale.reshape(1, 1, C), bn2_bias.reshape(1, 1, C),
                  f, leaky_first=False)
    return out


# speedup vs baseline: 1.2424x; 1.0685x over previous
import functools

import jax
import jax.numpy as jnp
from jax import lax
from jax.experimental import pallas as pl
from jax.experimental.pallas import tpu as pltpu
from jax.experimental.pallas import tpu_sc as plsc

N = 10000
C = 256
K = 27
NC = 512
DEPTH = 4

TN = 1000
NT = N // TN

_NCORE = 2
_NSUB = 16
_NW = _NCORE * _NSUB

_F32 = jnp.float32


def _sc_gather(table, idx, rows_per_iter):
    M = idx.shape[0]
    D = table.shape[1]
    b_per_w = M // _NW
    n_it = b_per_w // rows_per_iter
    assert M % _NW == 0 and b_per_w % rows_per_iter == 0
    assert rows_per_iter % 8 == 0

    mesh = plsc.VectorSubcoreMesh(core_axis_name="c", subcore_axis_name="s")
    R = rows_per_iter
    NB = 4

    @functools.partial(
        pl.kernel,
        mesh=mesh,
        out_type=jax.ShapeDtypeStruct((M, D), table.dtype),
        scratch_types=[
            pltpu.VMEM((b_per_w,), jnp.int32),
            [pltpu.VMEM((R, D), table.dtype) for _ in range(NB)],
            [pltpu.SemaphoreType.DMA for _ in range(NB)],
            [pltpu.SemaphoreType.DMA for _ in range(NB)],
        ],
    )
    def gather_kernel(table_hbm, idx_hbm, out_hbm, idx_v, bufs, gsems, ssems):
        wid = lax.axis_index("s") * _NCORE + lax.axis_index("c")
        base = wid * b_per_w
        pltpu.sync_copy(idx_hbm.at[pl.ds(base, b_per_w)], idx_v)

        def start_gather(i):
            return pltpu.async_copy(
                table_hbm.at[idx_v.at[pl.ds(i * R, R)]], bufs[i % NB],
                gsems[i % NB])

        def start_store(i):
            return pltpu.async_copy(
                bufs[i % NB], out_hbm.at[pl.ds(base + i * R, R)], ssems[i % NB])

        gathers = [None] * n_it
        stores = [None] * n_it
        for i in range(min(NB - 1, n_it)):
            gathers[i] = start_gather(i)
        for i in range(n_it):
            nxt = i + NB - 1
            if nxt < n_it:
                if nxt - NB >= 0:
                    stores[nxt - NB].wait()
                gathers[nxt] = start_gather(nxt)
            gathers[i].wait()
            stores[i] = start_store(i)
        for i in range(max(0, n_it - NB), n_it):
            stores[i].wait()

    return gather_kernel(table, idx)


def _leaky(x):
    return jnp.where(x >= 0, x, 0.01 * x)


def _onehot(cl, nc):
    io = lax.broadcasted_iota(jnp.int32, (cl.shape[0], nc), 1)
    return (cl[:, None] == io).astype(_F32)


def _dot(a, b):
    return jnp.dot(a, b, preferred_element_type=_F32)


def _dotT(a, b):
    return lax.dot_general(a, b, (((0,), (0,)), ((), ())),
                           preferred_element_type=_F32)


def _bn_from_stats(t, stats, scale, bias, n):
    mu = stats[0:1] / n
    var = stats[1:2] / n - mu * mu
    return (t - mu) / jnp.sqrt(var + 1e-5) * scale + bias


def _pack_bf16(t):
    tb = t.astype(jnp.bfloat16)
    lo = lax.bitcast_convert_type(tb[:, :128], jnp.uint16).astype(jnp.int32)
    hi = lax.bitcast_convert_type(tb[:, 128:], jnp.uint16).astype(jnp.int32)
    return lo | lax.shift_left(hi, 16)


def _unpack_bf16(p):
    lo = lax.bitcast_convert_type((p & 0xFFFF).astype(jnp.uint16), jnp.bfloat16)
    hi = lax.bitcast_convert_type(
        lax.shift_right_logical(p, 16).astype(jnp.uint16), jnp.bfloat16)
    return lo, hi


def _mm_stats(x, w):
    B, _, D = w.shape

    def body(x_ref, w_ref, raw_ref, st_ref):
        j = pl.program_id(1)
        t = _dot(x_ref[...], w_ref[0])
        raw_ref[0] = t
        s = jnp.sum(t, axis=0, keepdims=True)
        q = jnp.sum(t * t, axis=0, keepdims=True)
        st = jnp.concatenate([s, q], axis=0)

        @pl.when(j == 0)
        def _():
            st_ref[0] = st

        @pl.when(j > 0)
        def _():
            st_ref[0] += st

    return pl.pallas_call(
        body,
        grid=(B, NT),
        in_specs=[
            pl.BlockSpec((TN, C), lambda b, j: (j, 0)),
            pl.BlockSpec((1, C, D), lambda b, j: (b, 0, 0)),
        ],
        out_specs=[
            pl.BlockSpec((1, TN, D), lambda b, j: (b, j, 0)),
            pl.BlockSpec((1, 2, D), lambda b, j: (b, 0, 0)),
        ],
        out_shape=[
            jax.ShapeDtypeStruct((B, N, D), _F32),
            jax.ShapeDtypeStruct((B, 2, D), _F32),
        ],
    )(x, w)


def _apply_seg(raw, stats, scale, bias, cls):
    def body(raw_ref, st_ref, sc_ref, bi_ref, cl_ref, a_ref, seg_ref, cnt_ref):
        j = pl.program_id(1)
        a = _leaky(_bn_from_stats(raw_ref[0], st_ref[0], sc_ref[0], bi_ref[0], N))
        a_ref[0] = a
        oh = _onehot(cl_ref[0, 0, 0], NC)
        seg = _dotT(oh, a)
        cnt = jnp.sum(oh, axis=0, keepdims=True)

        @pl.when(j == 0)
        def _():
            seg_ref[0] = seg
            cnt_ref[0] = cnt

        @pl.when(j > 0)
        def _():
            seg_ref[0] += seg
            cnt_ref[0] += cnt

    return pl.pallas_call(
        body,
        grid=(3, NT),
        in_specs=[
            pl.BlockSpec((1, TN, C), lambda b, j: (b, j, 0)),
            pl.BlockSpec((1, 2, C), lambda b, j: (b, 0, 0)),
            pl.BlockSpec((1, 1, C), lambda b, j: (b, 0, 0)),
            pl.BlockSpec((1, 1, C), lambda b, j: (b, 0, 0)),
            pl.BlockSpec((1, 1, 1, TN), lambda b, j: (b, j, 0, 0)),
        ],
        out_specs=[
            pl.BlockSpec((1, TN, C), lambda b, j: (b, j, 0)),
            pl.BlockSpec((1, NC, C), lambda b, j: (b, 0, 0)),
            pl.BlockSpec((1, 1, NC), lambda b, j: (b, 0, 0)),
        ],
        out_shape=[
            jax.ShapeDtypeStruct((3, N, C), _F32),
            jax.ShapeDtypeStruct((3, NC, C), _F32),
            jax.ShapeDtypeStruct((3, 1, NC), _F32),
        ],
    )(raw, stats, scale, bias, cls)


def _center_mm_max(a, segA, cnt, cls, w):
    def body(a_ref, seg_ref, cnt_ref, cl_ref, w_ref, b_ref, mx_ref):
        j = pl.program_id(1)
        oh = _onehot(cl_ref[0, 0, 0], NC)
        rc = 1.0 / jnp.maximum(cnt_ref[0], 1.0)
        m = _dot(oh * rc, seg_ref[0])
        bt = _dot(a_ref[0] - m, w_ref[0])
        b_ref[0] = bt
        tm = jnp.max(bt)

        @pl.when(j == 0)
        def _():
            mx_ref[...] = jnp.full((1, 1, NC), tm, _F32)

        @pl.when(j > 0)
        def _():
            mx_ref[...] = jnp.maximum(mx_ref[...], tm)

    return pl.pallas_call(
        body,
        grid=(3, NT),
        in_specs=[
            pl.BlockSpec((1, TN, C), lambda b, j: (b, j, 0)),
            pl.BlockSpec((1, NC, C), lambda b, j: (b, 0, 0)),
            pl.BlockSpec((1, 1, NC), lambda b, j: (b, 0, 0)),
            pl.BlockSpec((1, 1, 1, TN), lambda b, j: (b, j, 0, 0)),
            pl.BlockSpec((1, C, C), lambda b, j: (b, 0, 0)),
        ],
        out_specs=[
            pl.BlockSpec((1, TN, C), lambda b, j: (b, j, 0)),
            pl.BlockSpec((1, 1, NC), lambda b, j: (b, 0, 0)),
        ],
        out_shape=[
            jax.ShapeDtypeStruct((3, N, C), _F32),
            jax.ShapeDtypeStruct((3, 1, NC), _F32),
        ],
    )(a, segA, cnt, cls, w)


def _exp_seg(bmm, bmax, cls):
    def body(b_ref, mx_ref, cl_ref, e_ref, seg_ref):
        j = pl.program_id(1)
        e = jnp.exp(b_ref[0] - mx_ref[0, 0, 0])
        e_ref[0] = e
        oh = _onehot(cl_ref[0, 0, 0], NC)
        seg = _dotT(oh, e)

        @pl.when(j == 0)
        def _():
            seg_ref[0] = seg

        @pl.when(j > 0)
        def _():
            seg_ref[0] += seg

    return pl.pallas_call(
        body,
        grid=(3, NT),
        in_specs=[
            pl.BlockSpec((1, TN, C), lambda b, j: (b, j, 0)),
            pl.BlockSpec((1, 1, NC), lambda b, j: (b, 0, 0)),
            pl.BlockSpec((1, 1, 1, TN), lambda b, j: (b, j, 0, 0)),
        ],
        out_specs=[
            pl.BlockSpec((1, TN, C), lambda b, j: (b, j, 0)),
            pl.BlockSpec((1, NC, C), lambda b, j: (b, 0, 0)),
        ],
        out_shape=[
            jax.ShapeDtypeStruct((3, N, C), _F32),
            jax.ShapeDtypeStruct((3, NC, C), _F32),
        ],
    )(bmm, bmax, cls)


def _pw_seg(praw, pstats, pscale, pbias, e, segE, cls):
    def body(p_ref, st_ref, sc_ref, bi_ref, e_ref, se_ref, cl_ref, seg_ref):
        j = pl.program_id(1)
        p = _leaky(_bn_from_stats(p_ref[0], st_ref[0], sc_ref[0], bi_ref[0], N))
        oh = _onehot(cl_ref[0, 0, 0], NC)
        den = _dot(oh, se_ref[0]) + 1e-6
        pp = p * (e_ref[0] / den)
        seg = _dotT(oh, pp)

        @pl.when(j == 0)
        def _():
            seg_ref[0] = seg

        @pl.when(j > 0)
        def _():
            seg_ref[0] += seg

    return pl.pallas_call(
        body,
        grid=(3, NT),
        in_specs=[
            pl.BlockSpec((1, TN, C), lambda b, j: (b, j, 0)),
            pl.BlockSpec((1, 2, C), lambda b, j: (b, 0, 0)),
            pl.BlockSpec((1, 1, C), lambda b, j: (b, 0, 0)),
            pl.BlockSpec((1, 1, C), lambda b, j: (b, 0, 0)),
            pl.BlockSpec((1, TN, C), lambda b, j: (b, j, 0)),
            pl.BlockSpec((1, NC, C), lambda b, j: (b, 0, 0)),
            pl.BlockSpec((1, 1, 1, TN), lambda b, j: (b, j, 0, 0)),
        ],
        out_specs=[
            pl.BlockSpec((1, NC, C), lambda b, j: (b, 0, 0)),
        ],
        out_shape=[
            jax.ShapeDtypeStruct((3, NC, C), _F32),
        ],
    )(praw, pstats, pscale, pbias, e, segE, cls)[0]


def _fuse(feat, awp, msk, praw3, pstats3, pscale3, pbias3, segP, cls3, fw1, fw2):
    def body(x_ref, aw_ref, mk_ref, p_ref, st_ref, sc_ref, bi_ref, sp_ref,
             cl_ref, f1_ref, f2_ref, fr_ref, fst_ref):
        j = pl.program_id(0)
        logits = _dot(x_ref[...], aw_ref[0]) + mk_ref[0]
        z = logits - jnp.max(logits, axis=1, keepdims=True)
        ez = jnp.exp(z)
        adp = ez / jnp.sum(ez, axis=1, keepdims=True)
        fsum = jnp.zeros((TN, C), _F32)
        for b in range(3):
            oh = _onehot(cl_ref[b, 0, 0], NC)
            pf = _dot(oh, sp_ref[b])
            fsum = fsum + adp[:, b:b + 1] * pf
        f4 = _leaky(_bn_from_stats(p_ref[0], st_ref[0], sc_ref[0], bi_ref[0], N))
        fr = _dot(f4, f1_ref[...]) + _dot(fsum, f2_ref[...])
        fr_ref[...] = fr
        s = jnp.sum(fr, axis=0, keepdims=True)
        q = jnp.sum(fr * fr, axis=0, keepdims=True)
        st = jnp.concatenate([s, q], axis=0)

        @pl.when(j == 0)
        def _():
            fst_ref[0] = st

        @pl.when(j > 0)
        def _():
            fst_ref[0] += st

    return pl.pallas_call(
        body,
        grid=(NT,),
        in_specs=[
            pl.BlockSpec((TN, C), lambda j: (j, 0)),
            pl.BlockSpec((1, C, 128), lambda j: (0, 0, 0)),
            pl.BlockSpec((1, 128), lambda j: (0, 0)),
            pl.BlockSpec((1, TN, C), lambda j: (3, j, 0)),
            pl.BlockSpec((1, 2, C), lambda j: (3, 0, 0)),
            pl.BlockSpec((1, 1, C), lambda j: (0, 0, 0)),
            pl.BlockSpec((1, 1, C), lambda j: (0, 0, 0)),
            pl.BlockSpec((3, NC, C), lambda j: (0, 0, 0)),
            pl.BlockSpec((3, 1, 1, TN), lambda j: (0, j, 0, 0)),
            pl.BlockSpec((C, C), lambda j: (0, 0)),
            pl.BlockSpec((C, C), lambda j: (0, 0)),
        ],
        out_specs=[
            pl.BlockSpec((TN, C), lambda j: (j, 0)),
            pl.BlockSpec((1, 2, C), lambda j: (0, 0, 0)),
        ],
        out_shape=[
            jax.ShapeDtypeStruct((N, C), _F32),
            jax.ShapeDtypeStruct((1, 2, C), _F32),
        ],
    )(feat, awp, msk, praw3, pstats3, pscale3, pbias3, segP, cls3, fw1, fw2)


def _bn_res(raw, stats, scale, bias, res, leaky_first, packed=False):
    def body(r_ref, st_ref, sc_ref, bi_ref, rs_ref, o_ref, *pk):
        t = _bn_from_stats(r_ref[...], st_ref[0], sc_ref[0], bi_ref[0], N)
        if leaky_first:
            t = _leaky(t) + rs_ref[...]
        else:
            t = _leaky(t + rs_ref[...])
        o_ref[...] = t
        if packed:
            pk[0][...] = _pack_bf16(t)

    out_specs = [pl.BlockSpec((TN, C), lambda j: (j, 0))]
    out_shape = [jax.ShapeDtypeStruct((N, C), _F32)]
    if packed:
        out_specs.append(pl.BlockSpec((TN, C // 2), lambda j: (j, 0)))
        out_shape.append(jax.ShapeDtypeStruct((N, C // 2), jnp.int32))
    r = pl.pallas_call(
        body,
        grid=(NT,),
        in_specs=[
            pl.BlockSpec((TN, C), lambda j: (j, 0)),
            pl.BlockSpec((1, 2, C), lambda j: (0, 0, 0)),
            pl.BlockSpec((1, 1, C), lambda j: (0, 0, 0)),
            pl.BlockSpec((1, 1, C), lambda j: (0, 0, 0)),
            pl.BlockSpec((TN, C), lambda j: (j, 0)),
        ],
        out_specs=out_specs,
        out_shape=out_shape,
    )(raw, stats, scale, bias, res)
    return r if packed else r[0]


def _bn_leaky_packed(raw, stats, scale, bias):
    def body(r_ref, st_ref, sc_ref, bi_ref, o_ref):
        t = _bn_from_stats(r_ref[...], st_ref[0], sc_ref[0], bi_ref[0], N)
        o_ref[...] = _pack_bf16(_leaky(t))

    return pl.pallas_call(
        body,
        grid=(NT,),
        in_specs=[
            pl.BlockSpec((TN, C), lambda j: (j, 0)),
            pl.BlockSpec((1, 2, C), lambda j: (0, 0, 0)),
            pl.BlockSpec((1, 1, C), lambda j: (0, 0, 0)),
            pl.BlockSpec((1, 1, C), lambda j: (0, 0, 0)),
        ],
        out_specs=[pl.BlockSpec((TN, C // 2), lambda j: (j, 0))],
        out_shape=[jax.ShapeDtypeStruct((N, C // 2), jnp.int32)],
    )(raw, stats, scale, bias)[0]


def _conv_part(g, w):
    kk = w.shape[0]

    def body(g_ref, w_ref, y_ref):
        k = pl.program_id(1)
        glo, ghi = _unpack_bf16(g_ref[...])
        t = (_dot(glo, w_ref[0, :C // 2]) + _dot(ghi, w_ref[0, C // 2:]))

        @pl.when(k == 0)
        def _():
            y_ref[...] = t

        @pl.when(k > 0)
        def _():
            y_ref[...] += t

    return pl.pallas_call(
        body,
        grid=(NT, kk),
        in_specs=[
            pl.BlockSpec((TN, C // 2), lambda j, k: (k * NT + j, 0)),
            pl.BlockSpec((1, C, C), lambda j, k: (k, 0, 0)),
        ],
        out_specs=[pl.BlockSpec((TN, C), lambda j, k: (j, 0))],
        out_shape=[jax.ShapeDtypeStruct((N, C), _F32)],
    )(g, w)[0]


def _conv(g, w, y_init):
    kk = w.shape[0]

    def body(g_ref, w_ref, yi_ref, y_ref, st_ref):
        j = pl.program_id(0)
        k = pl.program_id(1)
        glo, ghi = _unpack_bf16(g_ref[...])
        t = (_dot(glo, w_ref[0, :C // 2]) + _dot(ghi, w_ref[0, C // 2:]))

        @pl.when(k == 0)
        def _():
            y_ref[...] = yi_ref[...] + t

        @pl.when(k > 0)
        def _():
            y_ref[...] += t

        @pl.when(k == kk - 1)
        def _():
            y = y_ref[...]
            s = jnp.sum(y, axis=0, keepdims=True)
            q = jnp.sum(y * y, axis=0, keepdims=True)
            st = jnp.concatenate([s, q], axis=0)

            @pl.when(j == 0)
            def _():
                st_ref[0] = st

            @pl.when(j > 0)
            def _():
                st_ref[0] += st

    return pl.pallas_call(
        body,
        grid=(NT, kk),
        in_specs=[
            pl.BlockSpec((TN, C // 2), lambda j, k: (k * NT + j, 0)),
            pl.BlockSpec((1, C, C), lambda j, k: (k, 0, 0)),
            pl.BlockSpec((TN, C), lambda j, k: (j, 0)),
        ],
        out_specs=[
            pl.BlockSpec((TN, C), lambda j, k: (j, 0)),
            pl.BlockSpec((1, 2, C), lambda j, k: (0, 0, 0)),
        ],
        out_shape=[
            jax.ShapeDtypeStruct((N, C), _F32),
            jax.ShapeDtypeStruct((1, 2, C), _F32),
        ],
    )(g, w, y_init)


def kernel(feat, cluster0, cluster1, cluster2, neighbor_index, proj_w, proj_scale, proj_bias, lw_w, lw_scale, lw_bias, weight_w, adaptive_w, fuse_w, fuse_scale, fuse_bias, conv1_w, conv2_w, bn1_scale, bn1_bias, bn2_scale, bn2_bias):
    cls = jnp.stack([cluster0, cluster1, cluster2]).astype(jnp.int32)
    cls = cls.reshape(3, NT, 1, TN)

    a_raw, lw_st = _mm_stats(feat, lw_w)
    a, segA, cnt = _apply_seg(a_raw, lw_st,
                              lw_scale.reshape(3, 1, C), lw_bias.reshape(3, 1, C),
                              cls)
    bmm, bmax = _center_mm_max(a, segA, cnt, cls, weight_w)
    e, segE = _exp_seg(bmm, bmax, cls)

    p_raw, p_st = _mm_stats(feat, proj_w)
    segP = _pw_seg(p_raw[:3], p_st[:3],
                   proj_scale[:3].reshape(3, 1, C), proj_bias[:3].reshape(3, 1, C),
                   e, segE, cls)

    awp = jnp.pad(adaptive_w, ((0, 0), (0, 128 - (DEPTH - 1)))).reshape(1, C, 128)
    msk = jnp.where(jnp.arange(128) < DEPTH - 1, 0.0, -1e30)
    msk = msk.astype(_F32).reshape(1, 128)
    fuse_raw, fuse_st = _fuse(feat, awp, msk, p_raw, p_st,
                              proj_scale[3].reshape(1, 1, C),
                              proj_bias[3].reshape(1, 1, C),
                              segP, cls, fuse_w[:C], fuse_w[C:])
    f, f_pk = _bn_res(fuse_raw, fuse_st,
                      fuse_scale.reshape(1, 1, C), fuse_bias.reshape(1, 1, C),
                      feat, leaky_first=True, packed=True)

    MPAD = 288000
    nbrf = jnp.concatenate(
        [neighbor_index.reshape(-1).astype(jnp.int32),
         jnp.zeros((MPAD - K * N,), jnp.int32)]
    )
    c1w = conv1_w.astype(jnp.bfloat16)
    c2w = conv2_w.astype(jnp.bfloat16)

    KA = 16
    MA = KA * N
    nbrfA, nbrfB = nbrf[:MA], nbrf[MA:]

    g1a = _sc_gather(f_pk, nbrfA, 40)
    g1b = _sc_gather(f_pk, nbrfB, 40)
    y_a = _conv_part(g1a, c1w[:KA])
    y_raw, y_st = _conv(g1b, c1w[KA:], y_a)
    y_pk = _bn_leaky_packed(y_raw, y_st,
                            bn1_scale.reshape(1, 1, C), bn1_bias.reshape(1, 1, C))
    g2a = _sc_gather(y_pk, nbrfA, 40)
    g2b = _sc_gather(y_pk, nbrfB, 40)
    y2_a = _conv_part(g2a, c2w[:KA])
    y2_raw, y2_st = _conv(g2b, c2w[KA:], y2_a)
    out = _bn_res(y2_raw, y2_st,
                  bn2_scale.reshape(1, 1, C), bn2_bias.reshape(1, 1, C),
                  f, leaky_first=False)
    return out


# speedup vs baseline: 1.2849x; 1.0342x over previous
import functools

import jax
import jax.numpy as jnp
from jax import lax
from jax.experimental import pallas as pl
from jax.experimental.pallas import tpu as pltpu
from jax.experimental.pallas import tpu_sc as plsc

N = 10000
C = 256
K = 27
NC = 512
DEPTH = 4

TN = 1000
NT = N // TN
TNC = 2000
NTC = N // TNC

_NCORE = 2
_NSUB = 16
_NW = _NCORE * _NSUB

_F32 = jnp.float32


def _sc_gather(table, idx, rows_per_iter):
    M = idx.shape[0]
    D = table.shape[1]
    b_per_w = M // _NW
    n_it = b_per_w // rows_per_iter
    assert M % _NW == 0 and b_per_w % rows_per_iter == 0
    assert rows_per_iter % 8 == 0

    mesh = plsc.VectorSubcoreMesh(core_axis_name="c", subcore_axis_name="s")
    R = rows_per_iter
    NB = 4

    @functools.partial(
        pl.kernel,
        mesh=mesh,
        out_type=jax.ShapeDtypeStruct((M, D), table.dtype),
        scratch_types=[
            pltpu.VMEM((b_per_w,), jnp.int32),
            [pltpu.VMEM((R, D), table.dtype) for _ in range(NB)],
            [pltpu.SemaphoreType.DMA for _ in range(NB)],
            [pltpu.SemaphoreType.DMA for _ in range(NB)],
        ],
    )
    def gather_kernel(table_hbm, idx_hbm, out_hbm, idx_v, bufs, gsems, ssems):
        wid = lax.axis_index("s") * _NCORE + lax.axis_index("c")
        base = wid * b_per_w
        pltpu.sync_copy(idx_hbm.at[pl.ds(base, b_per_w)], idx_v)

        def start_gather(i):
            return pltpu.async_copy(
                table_hbm.at[idx_v.at[pl.ds(i * R, R)]], bufs[i % NB],
                gsems[i % NB])

        def start_store(i):
            return pltpu.async_copy(
                bufs[i % NB], out_hbm.at[pl.ds(base + i * R, R)], ssems[i % NB])

        gathers = [None] * n_it
        stores = [None] * n_it
        for i in range(min(NB - 1, n_it)):
            gathers[i] = start_gather(i)
        for i in range(n_it):
            nxt = i + NB - 1
            if nxt < n_it:
                if nxt - NB >= 0:
                    stores[nxt - NB].wait()
                gathers[nxt] = start_gather(nxt)
            gathers[i].wait()
            stores[i] = start_store(i)
        for i in range(max(0, n_it - NB), n_it):
            stores[i].wait()

    return gather_kernel(table, idx)


def _leaky(x):
    return jnp.where(x >= 0, x, 0.01 * x)


def _onehot(cl, nc):
    io = lax.broadcasted_iota(jnp.int32, (cl.shape[0], nc), 1)
    return (cl[:, None] == io).astype(_F32)


def _dot(a, b):
    return jnp.dot(a, b, preferred_element_type=_F32)


def _dotT(a, b):
    return lax.dot_general(a, b, (((0,), (0,)), ((), ())),
                           preferred_element_type=_F32)


def _bn_from_stats(t, stats, scale, bias, n):
    mu = stats[0:1] / n
    var = stats[1:2] / n - mu * mu
    return (t - mu) / jnp.sqrt(var + 1e-5) * scale + bias


def _pack_bf16(t):
    tb = t.astype(jnp.bfloat16)
    lo = lax.bitcast_convert_type(tb[:, :128], jnp.uint16).astype(jnp.int32)
    hi = lax.bitcast_convert_type(tb[:, 128:], jnp.uint16).astype(jnp.int32)
    return lo | lax.shift_left(hi, 16)


def _unpack_bf16(p):
    lo = lax.bitcast_convert_type((p & 0xFFFF).astype(jnp.uint16), jnp.bfloat16)
    hi = lax.bitcast_convert_type(
        lax.shift_right_logical(p, 16).astype(jnp.uint16), jnp.bfloat16)
    return lo, hi


def _mm_stats(x, w):
    B, _, D = w.shape

    def body(x_ref, w_ref, raw_ref, st_ref):
        j = pl.program_id(1)
        t = _dot(x_ref[...], w_ref[0])
        raw_ref[0] = t
        s = jnp.sum(t, axis=0, keepdims=True)
        q = jnp.sum(t * t, axis=0, keepdims=True)
        st = jnp.concatenate([s, q], axis=0)

        @pl.when(j == 0)
        def _():
            st_ref[0] = st

        @pl.when(j > 0)
        def _():
            st_ref[0] += st

    return pl.pallas_call(
        body,
        grid=(B, NT),
        in_specs=[
            pl.BlockSpec((TN, C), lambda b, j: (j, 0)),
            pl.BlockSpec((1, C, D), lambda b, j: (b, 0, 0)),
        ],
        out_specs=[
            pl.BlockSpec((1, TN, D), lambda b, j: (b, j, 0)),
            pl.BlockSpec((1, 2, D), lambda b, j: (b, 0, 0)),
        ],
        out_shape=[
            jax.ShapeDtypeStruct((B, N, D), _F32),
            jax.ShapeDtypeStruct((B, 2, D), _F32),
        ],
    )(x, w)


def _apply_seg(raw, stats, scale, bias, cls):
    def body(raw_ref, st_ref, sc_ref, bi_ref, cl_ref, a_ref, seg_ref, cnt_ref):
        j = pl.program_id(1)
        a = _leaky(_bn_from_stats(raw_ref[0], st_ref[0], sc_ref[0], bi_ref[0], N))
        a_ref[0] = a
        oh = _onehot(cl_ref[0, 0, 0], NC)
        seg = _dotT(oh, a)
        cnt = jnp.sum(oh, axis=0, keepdims=True)

        @pl.when(j == 0)
        def _():
            seg_ref[0] = seg
            cnt_ref[0] = cnt

        @pl.when(j > 0)
        def _():
            seg_ref[0] += seg
            cnt_ref[0] += cnt

    return pl.pallas_call(
        body,
        grid=(3, NT),
        in_specs=[
            pl.BlockSpec((1, TN, C), lambda b, j: (b, j, 0)),
            pl.BlockSpec((1, 2, C), lambda b, j: (b, 0, 0)),
            pl.BlockSpec((1, 1, C), lambda b, j: (b, 0, 0)),
            pl.BlockSpec((1, 1, C), lambda b, j: (b, 0, 0)),
            pl.BlockSpec((1, 1, 1, TN), lambda b, j: (b, j, 0, 0)),
        ],
        out_specs=[
            pl.BlockSpec((1, TN, C), lambda b, j: (b, j, 0)),
            pl.BlockSpec((1, NC, C), lambda b, j: (b, 0, 0)),
            pl.BlockSpec((1, 1, NC), lambda b, j: (b, 0, 0)),
        ],
        out_shape=[
            jax.ShapeDtypeStruct((3, N, C), _F32),
            jax.ShapeDtypeStruct((3, NC, C), _F32),
            jax.ShapeDtypeStruct((3, 1, NC), _F32),
        ],
    )(raw, stats, scale, bias, cls)


def _center_mm_max(a, segA, cnt, cls, w):
    def body(a_ref, seg_ref, cnt_ref, cl_ref, w_ref, b_ref, mx_ref):
        j = pl.program_id(1)
        oh = _onehot(cl_ref[0, 0, 0], NC)
        rc = 1.0 / jnp.maximum(cnt_ref[0], 1.0)
        m = _dot(oh * rc, seg_ref[0])
        bt = _dot(a_ref[0] - m, w_ref[0])
        b_ref[0] = bt
        tm = jnp.max(bt)

        @pl.when(j == 0)
        def _():
            mx_ref[...] = jnp.full((1, 1, NC), tm, _F32)

        @pl.when(j > 0)
        def _():
            mx_ref[...] = jnp.maximum(mx_ref[...], tm)

    return pl.pallas_call(
        body,
        grid=(3, NT),
        in_specs=[
            pl.BlockSpec((1, TN, C), lambda b, j: (b, j, 0)),
            pl.BlockSpec((1, NC, C), lambda b, j: (b, 0, 0)),
            pl.BlockSpec((1, 1, NC), lambda b, j: (b, 0, 0)),
            pl.BlockSpec((1, 1, 1, TN), lambda b, j: (b, j, 0, 0)),
            pl.BlockSpec((1, C, C), lambda b, j: (b, 0, 0)),
        ],
        out_specs=[
            pl.BlockSpec((1, TN, C), lambda b, j: (b, j, 0)),
            pl.BlockSpec((1, 1, NC), lambda b, j: (b, 0, 0)),
        ],
        out_shape=[
            jax.ShapeDtypeStruct((3, N, C), _F32),
            jax.ShapeDtypeStruct((3, 1, NC), _F32),
        ],
    )(a, segA, cnt, cls, w)


def _exp_seg(bmm, bmax, cls):
    def body(b_ref, mx_ref, cl_ref, e_ref, seg_ref):
        j = pl.program_id(1)
        e = jnp.exp(b_ref[0] - mx_ref[0, 0, 0])
        e_ref[0] = e
        oh = _onehot(cl_ref[0, 0, 0], NC)
        seg = _dotT(oh, e)

        @pl.when(j == 0)
        def _():
            seg_ref[0] = seg

        @pl.when(j > 0)
        def _():
            seg_ref[0] += seg

    return pl.pallas_call(
        body,
        grid=(3, NT),
        in_specs=[
            pl.BlockSpec((1, TN, C), lambda b, j: (b, j, 0)),
            pl.BlockSpec((1, 1, NC), lambda b, j: (b, 0, 0)),
            pl.BlockSpec((1, 1, 1, TN), lambda b, j: (b, j, 0, 0)),
        ],
        out_specs=[
            pl.BlockSpec((1, TN, C), lambda b, j: (b, j, 0)),
            pl.BlockSpec((1, NC, C), lambda b, j: (b, 0, 0)),
        ],
        out_shape=[
            jax.ShapeDtypeStruct((3, N, C), _F32),
            jax.ShapeDtypeStruct((3, NC, C), _F32),
        ],
    )(bmm, bmax, cls)


def _pw_seg(praw, pstats, pscale, pbias, e, segE, cls):
    def body(p_ref, st_ref, sc_ref, bi_ref, e_ref, se_ref, cl_ref, seg_ref):
        j = pl.program_id(1)
        p = _leaky(_bn_from_stats(p_ref[0], st_ref[0], sc_ref[0], bi_ref[0], N))
        oh = _onehot(cl_ref[0, 0, 0], NC)
        den = _dot(oh, se_ref[0]) + 1e-6
        pp = p * (e_ref[0] / den)
        seg = _dotT(oh, pp)

        @pl.when(j == 0)
        def _():
            seg_ref[0] = seg

        @pl.when(j > 0)
        def _():
            seg_ref[0] += seg

    return pl.pallas_call(
        body,
        grid=(3, NT),
        in_specs=[
            pl.BlockSpec((1, TN, C), lambda b, j: (b, j, 0)),
            pl.BlockSpec((1, 2, C), lambda b, j: (b, 0, 0)),
            pl.BlockSpec((1, 1, C), lambda b, j: (b, 0, 0)),
            pl.BlockSpec((1, 1, C), lambda b, j: (b, 0, 0)),
            pl.BlockSpec((1, TN, C), lambda b, j: (b, j, 0)),
            pl.BlockSpec((1, NC, C), lambda b, j: (b, 0, 0)),
            pl.BlockSpec((1, 1, 1, TN), lambda b, j: (b, j, 0, 0)),
        ],
        out_specs=[
            pl.BlockSpec((1, NC, C), lambda b, j: (b, 0, 0)),
        ],
        out_shape=[
            jax.ShapeDtypeStruct((3, NC, C), _F32),
        ],
    )(praw, pstats, pscale, pbias, e, segE, cls)[0]


def _fuse(feat, awp, msk, praw3, pstats3, pscale3, pbias3, segP, cls3, fw1, fw2):
    def body(x_ref, aw_ref, mk_ref, p_ref, st_ref, sc_ref, bi_ref, sp_ref,
             cl_ref, f1_ref, f2_ref, fr_ref, fst_ref):
        j = pl.program_id(0)
        logits = _dot(x_ref[...], aw_ref[0]) + mk_ref[0]
        z = logits - jnp.max(logits, axis=1, keepdims=True)
        ez = jnp.exp(z)
        adp = ez / jnp.sum(ez, axis=1, keepdims=True)
        fsum = jnp.zeros((TN, C), _F32)
        for b in range(3):
            oh = _onehot(cl_ref[b, 0, 0], NC)
            pf = _dot(oh, sp_ref[b])
            fsum = fsum + adp[:, b:b + 1] * pf
        f4 = _leaky(_bn_from_stats(p_ref[0], st_ref[0], sc_ref[0], bi_ref[0], N))
        fr = _dot(f4, f1_ref[...]) + _dot(fsum, f2_ref[...])
        fr_ref[...] = fr
        s = jnp.sum(fr, axis=0, keepdims=True)
        q = jnp.sum(fr * fr, axis=0, keepdims=True)
        st = jnp.concatenate([s, q], axis=0)

        @pl.when(j == 0)
        def _():
            fst_ref[0] = st

        @pl.when(j > 0)
        def _():
            fst_ref[0] += st

    return pl.pallas_call(
        body,
        grid=(NT,),
        in_specs=[
            pl.BlockSpec((TN, C), lambda j: (j, 0)),
            pl.BlockSpec((1, C, 128), lambda j: (0, 0, 0)),
            pl.BlockSpec((1, 128), lambda j: (0, 0)),
            pl.BlockSpec((1, TN, C), lambda j: (3, j, 0)),
            pl.BlockSpec((1, 2, C), lambda j: (3, 0, 0)),
            pl.BlockSpec((1, 1, C), lambda j: (0, 0, 0)),
            pl.BlockSpec((1, 1, C), lambda j: (0, 0, 0)),
            pl.BlockSpec((3, NC, C), lambda j: (0, 0, 0)),
            pl.BlockSpec((3, 1, 1, TN), lambda j: (0, j, 0, 0)),
            pl.BlockSpec((C, C), lambda j: (0, 0)),
            pl.BlockSpec((C, C), lambda j: (0, 0)),
        ],
        out_specs=[
            pl.BlockSpec((TN, C), lambda j: (j, 0)),
            pl.BlockSpec((1, 2, C), lambda j: (0, 0, 0)),
        ],
        out_shape=[
            jax.ShapeDtypeStruct((N, C), _F32),
            jax.ShapeDtypeStruct((1, 2, C), _F32),
        ],
    )(feat, awp, msk, praw3, pstats3, pscale3, pbias3, segP, cls3, fw1, fw2)


def _bn_res(raw, stats, scale, bias, res, leaky_first, packed=False):
    def body(r_ref, st_ref, sc_ref, bi_ref, rs_ref, o_ref, *pk):
        t = _bn_from_stats(r_ref[...], st_ref[0], sc_ref[0], bi_ref[0], N)
        if leaky_first:
            t = _leaky(t) + rs_ref[...]
        else:
            t = _leaky(t + rs_ref[...])
        o_ref[...] = t
        if packed:
            pk[0][...] = _pack_bf16(t)

    out_specs = [pl.BlockSpec((TN, C), lambda j: (j, 0))]
    out_shape = [jax.ShapeDtypeStruct((N, C), _F32)]
    if packed:
        out_specs.append(pl.BlockSpec((TN, C // 2), lambda j: (j, 0)))
        out_shape.append(jax.ShapeDtypeStruct((N, C // 2), jnp.int32))
    r = pl.pallas_call(
        body,
        grid=(NT,),
        in_specs=[
            pl.BlockSpec((TN, C), lambda j: (j, 0)),
            pl.BlockSpec((1, 2, C), lambda j: (0, 0, 0)),
            pl.BlockSpec((1, 1, C), lambda j: (0, 0, 0)),
            pl.BlockSpec((1, 1, C), lambda j: (0, 0, 0)),
            pl.BlockSpec((TN, C), lambda j: (j, 0)),
        ],
        out_specs=out_specs,
        out_shape=out_shape,
    )(raw, stats, scale, bias, res)
    return r if packed else r[0]


def _bn_leaky_packed(raw, stats, scale, bias):
    def body(r_ref, st_ref, sc_ref, bi_ref, o_ref):
        t = _bn_from_stats(r_ref[...], st_ref[0], sc_ref[0], bi_ref[0], N)
        o_ref[...] = _pack_bf16(_leaky(t))

    return pl.pallas_call(
        body,
        grid=(NT,),
        in_specs=[
            pl.BlockSpec((TN, C), lambda j: (j, 0)),
            pl.BlockSpec((1, 2, C), lambda j: (0, 0, 0)),
            pl.BlockSpec((1, 1, C), lambda j: (0, 0, 0)),
            pl.BlockSpec((1, 1, C), lambda j: (0, 0, 0)),
        ],
        out_specs=[pl.BlockSpec((TN, C // 2), lambda j: (j, 0))],
        out_shape=[jax.ShapeDtypeStruct((N, C // 2), jnp.int32)],
    )(raw, stats, scale, bias)[0]


def _conv_part(g, w):
    kk = w.shape[0]

    def body(g_ref, w_ref, y_ref):
        k = pl.program_id(1)
        glo, ghi = _unpack_bf16(g_ref[...])
        t = (_dot(glo, w_ref[0, :C // 2]) + _dot(ghi, w_ref[0, C // 2:]))

        @pl.when(k == 0)
        def _():
            y_ref[...] = t

        @pl.when(k > 0)
        def _():
            y_ref[...] += t

    return pl.pallas_call(
        body,
        grid=(NTC, kk),
        in_specs=[
            pl.BlockSpec((TNC, C // 2), lambda j, k: (k * NTC + j, 0)),
            pl.BlockSpec((1, C, C), lambda j, k: (k, 0, 0)),
        ],
        out_specs=[pl.BlockSpec((TNC, C), lambda j, k: (j, 0))],
        out_shape=[jax.ShapeDtypeStruct((N, C), _F32)],
    )(g, w)[0]


def _conv(g, w, y_init):
    kk = w.shape[0]

    def body(g_ref, w_ref, yi_ref, y_ref, st_ref):
        j = pl.program_id(0)
        k = pl.program_id(1)
        glo, ghi = _unpack_bf16(g_ref[...])
        t = (_dot(glo, w_ref[0, :C // 2]) + _dot(ghi, w_ref[0, C // 2:]))

        @pl.when(k == 0)
        def _():
            y_ref[...] = yi_ref[...] + t

        @pl.when(k > 0)
        def _():
            y_ref[...] += t

        @pl.when(k == kk - 1)
        def _():
            y = y_ref[...]
            s = jnp.sum(y, axis=0, keepdims=True)
            q = jnp.sum(y * y, axis=0, keepdims=True)
            st = jnp.concatenate([s, q], axis=0)

            @pl.when(j == 0)
            def _():
                st_ref[0] = st

            @pl.when(j > 0)
            def _():
                st_ref[0] += st

    return pl.pallas_call(
        body,
        grid=(NTC, kk),
        in_specs=[
            pl.BlockSpec((TNC, C // 2), lambda j, k: (k * NTC + j, 0)),
            pl.BlockSpec((1, C, C), lambda j, k: (k, 0, 0)),
            pl.BlockSpec((TNC, C), lambda j, k: (j, 0)),
        ],
        out_specs=[
            pl.BlockSpec((TNC, C), lambda j, k: (j, 0)),
            pl.BlockSpec((1, 2, C), lambda j, k: (0, 0, 0)),
        ],
        out_shape=[
            jax.ShapeDtypeStruct((N, C), _F32),
            jax.ShapeDtypeStruct((1, 2, C), _F32),
        ],
    )(g, w, y_init)


def kernel(feat, cluster0, cluster1, cluster2, neighbor_index, proj_w, proj_scale, proj_bias, lw_w, lw_scale, lw_bias, weight_w, adaptive_w, fuse_w, fuse_scale, fuse_bias, conv1_w, conv2_w, bn1_scale, bn1_bias, bn2_scale, bn2_bias):
    cls = jnp.stack([cluster0, cluster1, cluster2]).astype(jnp.int32)
    cls = cls.reshape(3, NT, 1, TN)

    a_raw, lw_st = _mm_stats(feat, lw_w)
    a, segA, cnt = _apply_seg(a_raw, lw_st,
                              lw_scale.reshape(3, 1, C), lw_bias.reshape(3, 1, C),
                              cls)
    bmm, bmax = _center_mm_max(a, segA, cnt, cls, weight_w)
    e, segE = _exp_seg(bmm, bmax, cls)

    p_raw, p_st = _mm_stats(feat, proj_w)
    segP = _pw_seg(p_raw[:3], p_st[:3],
                   proj_scale[:3].reshape(3, 1, C), proj_bias[:3].reshape(3, 1, C),
                   e, segE, cls)

    awp = jnp.pad(adaptive_w, ((0, 0), (0, 128 - (DEPTH - 1)))).reshape(1, C, 128)
    msk = jnp.where(jnp.arange(128) < DEPTH - 1, 0.0, -1e30)
    msk = msk.astype(_F32).reshape(1, 128)
    fuse_raw, fuse_st = _fuse(feat, awp, msk, p_raw, p_st,
                              proj_scale[3].reshape(1, 1, C),
                              proj_bias[3].reshape(1, 1, C),
                              segP, cls, fuse_w[:C], fuse_w[C:])
    f, f_pk = _bn_res(fuse_raw, fuse_st,
                      fuse_scale.reshape(1, 1, C), fuse_bias.reshape(1, 1, C),
                      feat, leaky_first=True, packed=True)

    MPAD = 288000
    nbrf = jnp.concatenate(
        [neighbor_index.reshape(-1).astype(jnp.int32),
         jnp.zeros((MPAD - K * N,), jnp.int32)]
    )
    c1w = conv1_w.astype(jnp.bfloat16)
    c2w = conv2_w.astype(jnp.bfloat16)

    KA = 16
    MA = KA * N
    nbrfA, nbrfB = nbrf[:MA], nbrf[MA:]

    g1a = _sc_gather(f_pk, nbrfA, 40)
    g1b = _sc_gather(f_pk, nbrfB, 40)
    y_a = _conv_part(g1a, c1w[:KA])
    y_raw, y_st = _conv(g1b, c1w[KA:], y_a)
    y_pk = _bn_leaky_packed(y_raw, y_st,
                            bn1_scale.reshape(1, 1, C), bn1_bias.reshape(1, 1, C))
    g2a = _sc_gather(y_pk, nbrfA, 40)
    g2b = _sc_gather(y_pk, nbrfB, 40)
    y2_a = _conv_part(g2a, c2w[:KA])
    y2_raw, y2_st = _conv(g2b, c2w[KA:], y2_a)
    out = _bn_res(y2_raw, y2_st,
                  bn2_scale.reshape(1, 1, C), bn2_bias.reshape(1, 1, C),
                  f, leaky_first=False)
    return out
